# Initial kernel scaffold; baseline (speedup 1.0000x reference)
import jax
import jax.numpy as jnp
from jax.experimental import pallas as pl


def kernel(x, pos, pe, params, edge_index, batch):
    raise NotImplementedError("write your pallas kernel here")


# speedup vs baseline: 2.7387x; 2.7387x over previous
import functools

import jax
import jax.numpy as jnp
from jax import lax
from jax.experimental import pallas as pl
from jax.experimental.pallas import tpu as pltpu
from jax.experimental.pallas import tpu_sc as plsc

NC = 2
NS = 16
NW = NC * NS
LANE = 16

H = 128


def _mesh():
    return plsc.VectorSubcoreMesh(core_axis_name="c", subcore_axis_name="s",
                                  num_cores=NC, num_subcores=NS)


def _sc_d2(pxa, pya, pza, send, rec):
    N = pxa.shape[0]
    E = send.shape[0]
    EW = E // NW
    C = 80
    steps = EW // C

    @functools.partial(
        pl.kernel,
        out_type=jax.ShapeDtypeStruct((E,), jnp.float32),
        mesh=_mesh(),
        compiler_params=pltpu.CompilerParams(needs_layout_passes=False),
        scratch_types=[
            pltpu.VMEM((N,), jnp.float32),
            pltpu.VMEM((N,), jnp.float32),
            pltpu.VMEM((N,), jnp.float32),
            pltpu.VMEM((C,), jnp.int32),
            pltpu.VMEM((C,), jnp.int32),
            pltpu.VMEM((C,), jnp.float32),
        ],
    )
    def k(px_hbm, py_hbm, pz_hbm, send_hbm, rec_hbm, d2_hbm,
          px, py, pz, sidx, ridx, dbuf):
        wid = lax.axis_index("s") * NC + lax.axis_index("c")
        base = wid * EW
        pltpu.sync_copy(px_hbm, px)
        pltpu.sync_copy(py_hbm, py)
        pltpu.sync_copy(pz_hbm, pz)

        def chunk(i, carry):
            off = base + i * C
            pltpu.sync_copy(send_hbm.at[pl.ds(off, C)], sidx)
            pltpu.sync_copy(rec_hbm.at[pl.ds(off, C)], ridx)

            def sub(j, carry2):
                s16 = sidx[pl.ds(j * LANE, LANE)]
                r16 = ridx[pl.ds(j * LANE, LANE)]
                acc = jnp.zeros((LANE,), jnp.float32)
                for tab in (px, py, pz):
                    a = plsc.load_gather(tab, [s16])
                    b = plsc.load_gather(tab, [r16])
                    d = a - b
                    acc = acc + d * d
                dbuf[pl.ds(j * LANE, LANE)] = acc
                return carry2

            lax.fori_loop(0, C // LANE, sub, 0)
            pltpu.sync_copy(dbuf, d2_hbm.at[pl.ds(off, C)])
            return carry

        lax.fori_loop(0, steps, chunk, 0)

    return k(pxa, pya, pza, send, rec)


def _sc_gather(T2, U2, send, rec):
    N, D = T2.shape
    E = send.shape[0]
    EW = E // NW
    C = 80
    steps = EW // C

    @functools.partial(
        pl.kernel,
        out_type=(jax.ShapeDtypeStruct((E, D), jnp.float32),
                  jax.ShapeDtypeStruct((E, D), jnp.float32)),
        mesh=_mesh(),
        compiler_params=pltpu.CompilerParams(needs_layout_passes=False),
        scratch_types=[
            pltpu.VMEM((C,), jnp.int32),
            pltpu.VMEM((C,), jnp.int32),
            pltpu.VMEM((C, D), jnp.float32),
            pltpu.VMEM((C, D), jnp.float32),
            pltpu.SemaphoreType.DMA,
            pltpu.SemaphoreType.DMA,
        ],
    )
    def k(t_hbm, u_hbm, send_hbm, rec_hbm, g1_hbm, g2_hbm,
          sidx, ridx, rows1, rows2, sem1, sem2):
        wid = lax.axis_index("s") * NC + lax.axis_index("c")
        base = wid * EW

        def chunk(i, carry):
            off = base + i * C
            pltpu.sync_copy(send_hbm.at[pl.ds(off, C)], sidx)
            pltpu.sync_copy(rec_hbm.at[pl.ds(off, C)], ridx)
            cp1 = pltpu.async_copy(t_hbm.at[sidx], rows1, sem1)
            cp2 = pltpu.async_copy(u_hbm.at[ridx], rows2, sem2)
            cp1.wait()
            cp2.wait()
            pltpu.sync_copy(rows1, g1_hbm.at[pl.ds(off, C)])
            pltpu.sync_copy(rows2, g2_hbm.at[pl.ds(off, C)])
            return carry

        lax.fori_loop(0, steps, chunk, 0)

    return k(T2, U2, send, rec)


def _sc_scatter(M, rec, zeros):
    _, E, D = M.shape
    NP = zeros.shape[0]
    ET = E // NS
    C = 80
    steps = ET // C
    NT = NP // NS

    @functools.partial(
        pl.kernel,
        out_type=jax.ShapeDtypeStruct((2, NP, D), jnp.float32),
        mesh=_mesh(),
        compiler_params=pltpu.CompilerParams(needs_layout_passes=False),
        scratch_types=[
            pltpu.VMEM((C,), jnp.int32),
            pltpu.VMEM((C, D), jnp.float32),
            pltpu.VMEM_SHARED((NP, D), jnp.float32),
            pltpu.SemaphoreType.DMA,
        ],
    )
    def k(m_hbm, rec_hbm, z_hbm, out_hbm, ridx, rows, acc, sem):
        cid = lax.axis_index("c")
        sid = lax.axis_index("s")
        nb = pl.multiple_of(sid * NT, 8)
        pltpu.sync_copy(z_hbm.at[pl.ds(nb, NT)], acc.at[pl.ds(nb, NT)])
        plsc.subcore_barrier()

        base = sid * ET

        def chunk(i, carry):
            off = base + i * C
            pltpu.sync_copy(rec_hbm.at[pl.ds(off, C)], ridx)
            cp = pltpu.async_copy(m_hbm.at[cid, pl.ds(off, C)], rows, sem)
            cp.wait()
            pltpu.sync_copy(rows, acc.at[ridx], add=True)
            return carry

        lax.fori_loop(0, steps, chunk, 0)
        plsc.subcore_barrier()
        pltpu.sync_copy(acc.at[pl.ds(nb, NT)],
                        out_hbm.at[cid, pl.ds(nb, NT)])

    return k(M, rec, zeros)


_silu = jax.nn.silu


def _tc_embed(x, pe, E1w, E1b, E2w, E2b, G1w, G1b, G2w, G2b):
    N = x.shape[0]
    BN = 2000

    def body(x_ref, pe_ref, e1w, e1b, e2w, e2b, g1w, g1b, g2w, g2b,
             h_ref, peh_ref):
        xin = jnp.concatenate([x_ref[...], pe_ref[...]], axis=1)
        t = _silu(jnp.dot(xin, e1w[...], preferred_element_type=jnp.float32)
                  + e1b[...])
        h_ref[...] = jnp.dot(t, e2w[...], preferred_element_type=jnp.float32) + e2b[...]
        tp = _silu(jnp.dot(pe_ref[...], g1w[...], preferred_element_type=jnp.float32)
                   + g1b[...])
        peh_ref[...] = jnp.dot(tp, g2w[...], preferred_element_type=jnp.float32) + g2b[...]

    full = lambda s: pl.BlockSpec(s, lambda i: (0, 0))
    return pl.pallas_call(
        body,
        grid=(N // BN,),
        in_specs=[
            pl.BlockSpec((BN, x.shape[1]), lambda i: (i, 0)),
            pl.BlockSpec((BN, pe.shape[1]), lambda i: (i, 0)),
            full(E1w.shape), full((1, H)), full(E2w.shape), full((1, H)),
            full(G1w.shape), full((1, H)), full(G2w.shape), full((1, H)),
        ],
        out_specs=[pl.BlockSpec((BN, H), lambda i: (i, 0)),
                   pl.BlockSpec((BN, H), lambda i: (i, 0))],
        out_shape=[jax.ShapeDtypeStruct((N, H), jnp.float32),
                   jax.ShapeDtypeStruct((N, H), jnp.float32)],
    )(x, pe, E1w, E1b[None, :], E2w, E2b[None, :],
      G1w, G1b[None, :], G2w, G2b[None, :])


def _tc_proj(h, pe_h, WT, WU):
    N = h.shape[0]
    BN = 2000

    def body(h_ref, pe_ref, wt, wu, t_ref, u_ref):
        z = jnp.concatenate([h_ref[...], pe_ref[...]], axis=1)
        t_ref[...] = jnp.dot(z, wt[...], preferred_element_type=jnp.float32)
        u_ref[...] = jnp.dot(z, wu[...], preferred_element_type=jnp.float32)

    return pl.pallas_call(
        body,
        grid=(N // BN,),
        in_specs=[
            pl.BlockSpec((BN, H), lambda i: (i, 0)),
            pl.BlockSpec((BN, H), lambda i: (i, 0)),
            pl.BlockSpec(WT.shape, lambda i: (0, 0)),
            pl.BlockSpec(WU.shape, lambda i: (0, 0)),
        ],
        out_specs=[pl.BlockSpec((BN, 2 * H), lambda i: (i, 0)),
                   pl.BlockSpec((BN, 2 * H), lambda i: (i, 0))],
        out_shape=[jax.ShapeDtypeStruct((N, 2 * H), jnp.float32),
                   jax.ShapeDtypeStruct((N, 2 * H), jnp.float32)],
    )(h, pe_h, WT, WU)


def _tc_edge(g1, g2, d2, vecs, W2, W2p):
    E = g1.shape[0]
    BE = 2000

    def body(g1_ref, g2_ref, d2_ref, v_ref, w2, w2p, m_ref):
        dist = jnp.sqrt(d2_ref[...])
        g = g1_ref[...] + g2_ref[...]
        pre1 = g[:, :H] + dist * v_ref[0:1, :] + v_ref[1:2, :]
        pre1p = g[:, H:] + dist * v_ref[2:3, :] + v_ref[3:4, :]
        t = _silu(pre1)
        u = jnp.dot(t, w2[...], preferred_element_type=jnp.float32) + v_ref[4:5, :]
        m_ref[0] = _silu(u)
        tp = jnp.tanh(pre1p)
        up = jnp.dot(tp, w2p[...], preferred_element_type=jnp.float32) + v_ref[5:6, :]
        m_ref[1] = jnp.tanh(up)

    return pl.pallas_call(
        body,
        grid=(E // BE,),
        in_specs=[
            pl.BlockSpec((BE, 2 * H), lambda i: (i, 0)),
            pl.BlockSpec((BE, 2 * H), lambda i: (i, 0)),
            pl.BlockSpec((BE, 1), lambda i: (i, 0)),
            pl.BlockSpec(vecs.shape, lambda i: (0, 0)),
            pl.BlockSpec(W2.shape, lambda i: (0, 0)),
            pl.BlockSpec(W2p.shape, lambda i: (0, 0)),
        ],
        out_specs=pl.BlockSpec((2, BE, H), lambda i: (0, i, 0)),
        out_shape=jax.ShapeDtypeStruct((2, E, H), jnp.float32),
    )(g1, g2, d2, vecs, W2, W2p)


def _tc_update(h, pe_h, aggr, aggrp, V1, c1, V2, c2, P1, p1, P2, p2):
    N = h.shape[0]
    BN = 2000

    def body(h_ref, pe_ref, a_ref, ap_ref, v1, c1r, v2, c2r, q1, p1r, q2, p2r,
             hn_ref, pen_ref):
        cat = jnp.concatenate([h_ref[...], pe_ref[...], a_ref[...]], axis=1)
        z = _silu(jnp.dot(cat, v1[...], preferred_element_type=jnp.float32) + c1r[...])
        upd = jnp.dot(z, v2[...], preferred_element_type=jnp.float32) + c2r[...]
        hn_ref[...] = h_ref[...] + upd
        catp = jnp.concatenate([pe_ref[...], ap_ref[...]], axis=1)
        zp = jnp.tanh(jnp.dot(catp, q1[...], preferred_element_type=jnp.float32) + p1r[...])
        updp = jnp.tanh(jnp.dot(zp, q2[...], preferred_element_type=jnp.float32) + p2r[...])
        pen_ref[...] = pe_ref[...] + updp

    full = lambda s: pl.BlockSpec(s, lambda i: (0, 0))
    return pl.pallas_call(
        body,
        grid=(N // BN,),
        in_specs=[
            pl.BlockSpec((BN, H), lambda i: (i, 0)),
            pl.BlockSpec((BN, H), lambda i: (i, 0)),
            pl.BlockSpec((BN, H), lambda i: (i, 0)),
            pl.BlockSpec((BN, H), lambda i: (i, 0)),
            full(V1.shape), full((1, H)), full(V2.shape), full((1, H)),
            full(P1.shape), full((1, H)), full(P2.shape), full((1, H)),
        ],
        out_specs=[pl.BlockSpec((BN, H), lambda i: (i, 0)),
                   pl.BlockSpec((BN, H), lambda i: (i, 0))],
        out_shape=[jax.ShapeDtypeStruct((N, H), jnp.float32),
                   jax.ShapeDtypeStruct((N, H), jnp.float32)],
    )(h, pe_h, aggr, aggrp, V1, c1[None, :], V2, c2[None, :],
      P1, p1[None, :], P2, p2[None, :])


def _tc_final(h, batch2d, NB, Q1, q1, Q2, q2, R1, r1, R2p, r2p):
    N = h.shape[0]

    def body(h_ref, b_ref, w1, b1r, w2, b2r, w3, b3r, w4, b4r, out_ref):
        t = _silu(jnp.dot(h_ref[...], w1[...], preferred_element_type=jnp.float32)
                  + b1r[...])
        hpre = jnp.dot(t, w2[...], preferred_element_type=jnp.float32) + b2r[...]
        seg = lax.broadcasted_iota(jnp.int32, (NB, N), 0)
        oh = (b_ref[...] == seg).astype(jnp.float32)
        pooled = jnp.dot(oh, hpre, preferred_element_type=jnp.float32)
        tr = _silu(jnp.dot(pooled, w3[...], preferred_element_type=jnp.float32)
                   + b3r[...])
        out_ref[...] = jnp.dot(tr, w4[...], preferred_element_type=jnp.float32) + b4r[...]

    return pl.pallas_call(
        body,
        out_shape=jax.ShapeDtypeStruct((NB, H), jnp.float32),
    )(h, batch2d, Q1, q1[None, :], Q2, q2[None, :],
      R1, r1[None, :], R2p, r2p[None, :])


def kernel(x, pos, pe, params, edge_index, batch):
    N = x.shape[0]
    E = edge_index.shape[1]
    NB = 64

    send = edge_index[0].astype(jnp.int32)
    rec = edge_index[1].astype(jnp.int32)

    d2 = _sc_d2(pos[:, 0], pos[:, 1], pos[:, 2], send, rec)
    d2c = d2[:, None]

    emb = params['embed']
    embp = params['embed_pe']
    h, pe_h = _tc_embed(x, pe,
                        emb[0]['w'], emb[0]['b'], emb[1]['w'], emb[1]['b'],
                        embp[0]['w'], embp[0]['b'], embp[1]['w'], embp[1]['b'])

    NPAD = 10240
    zeros_nh = jnp.zeros((NPAD, H), jnp.float32)

    for lp in params['layers']:
        W1 = lp['message_mlp'][0]['w']
        b1 = lp['message_mlp'][0]['b']
        W2 = lp['message_mlp'][1]['w']
        b2 = lp['message_mlp'][1]['b']
        Wp1 = lp['message_pos_mlp'][0]['w']
        bp1 = lp['message_pos_mlp'][0]['b']
        Wp2 = lp['message_pos_mlp'][1]['w']
        bp2 = lp['message_pos_mlp'][1]['b']

        zblk = jnp.zeros((H, H), jnp.float32)
        WT = jnp.concatenate([
            jnp.concatenate([W1[:H], zblk], axis=1),
            jnp.concatenate([W1[H:2 * H], Wp1[:H]], axis=1)], axis=0)
        WU = jnp.concatenate([
            jnp.concatenate([W1[2 * H:3 * H], zblk], axis=1),
            jnp.concatenate([W1[3 * H:4 * H], Wp1[H:2 * H]], axis=1)], axis=0)
        vecs = jnp.stack([W1[4 * H], b1, Wp1[2 * H], bp1, b2, bp2,
                          jnp.zeros((H,), jnp.float32),
                          jnp.zeros((H,), jnp.float32)], axis=0)

        T2, U2 = _tc_proj(h, pe_h, WT, WU)
        g1, g2 = _sc_gather(T2, U2, send, rec)
        M = _tc_edge(g1, g2, d2c, vecs, W2, Wp2)
        A = _sc_scatter(M, rec, zeros_nh)

        h, pe_h = _tc_update(h, pe_h, A[0, :N], A[1, :N],
                             lp['update_mlp'][0]['w'], lp['update_mlp'][0]['b'],
                             lp['update_mlp'][1]['w'], lp['update_mlp'][1]['b'],
                             lp['update_pos_mlp'][0]['w'], lp['update_pos_mlp'][0]['b'],
                             lp['update_pos_mlp'][1]['w'], lp['update_pos_mlp'][1]['b'])

    pr = params['pre_readout']
    ro = params['readout']
    R2 = ro[1]['w']
    R2p = jnp.concatenate([R2, jnp.zeros((H, H - 1), jnp.float32)], axis=1)
    r2p = jnp.concatenate([ro[1]['b'], jnp.zeros((H - 1,), jnp.float32)], axis=0)
    batch2d = batch.astype(jnp.int32)[None, :]
    out = _tc_final(h, batch2d, NB,
                    pr[0]['w'], pr[0]['b'], pr[1]['w'], pr[1]['b'],
                    ro[0]['w'], ro[0]['b'], R2p, r2p)
    return out[:, 0]


# speedup vs baseline: 3.2926x; 1.2022x over previous
import functools

import jax
import jax.numpy as jnp
from jax import lax
from jax.experimental import pallas as pl
from jax.experimental.pallas import tpu as pltpu
from jax.experimental.pallas import tpu_sc as plsc

NC = 2
NS = 16
NW = NC * NS
LANE = 16

H = 128


def _mesh():
    return plsc.VectorSubcoreMesh(core_axis_name="c", subcore_axis_name="s",
                                  num_cores=NC, num_subcores=NS)


def _sc_d2(pxa, pya, pza, send, rec):
    N = pxa.shape[0]
    E = send.shape[0]
    EW = E // NW
    C = 80
    steps = EW // C

    @functools.partial(
        pl.kernel,
        out_type=jax.ShapeDtypeStruct((E,), jnp.float32),
        mesh=_mesh(),
        compiler_params=pltpu.CompilerParams(needs_layout_passes=False),
        scratch_types=[
            pltpu.VMEM((N,), jnp.float32),
            pltpu.VMEM((N,), jnp.float32),
            pltpu.VMEM((N,), jnp.float32),
            pltpu.VMEM((C,), jnp.int32),
            pltpu.VMEM((C,), jnp.int32),
            pltpu.VMEM((C,), jnp.float32),
        ],
    )
    def k(px_hbm, py_hbm, pz_hbm, send_hbm, rec_hbm, d2_hbm,
          px, py, pz, sidx, ridx, dbuf):
        wid = lax.axis_index("s") * NC + lax.axis_index("c")
        base = wid * EW
        pltpu.sync_copy(px_hbm, px)
        pltpu.sync_copy(py_hbm, py)
        pltpu.sync_copy(pz_hbm, pz)

        def chunk(i, carry):
            off = base + i * C
            pltpu.sync_copy(send_hbm.at[pl.ds(off, C)], sidx)
            pltpu.sync_copy(rec_hbm.at[pl.ds(off, C)], ridx)

            def sub(j, carry2):
                s16 = sidx[pl.ds(j * LANE, LANE)]
                r16 = ridx[pl.ds(j * LANE, LANE)]
                acc = jnp.zeros((LANE,), jnp.float32)
                for tab in (px, py, pz):
                    a = plsc.load_gather(tab, [s16])
                    b = plsc.load_gather(tab, [r16])
                    d = a - b
                    acc = acc + d * d
                dbuf[pl.ds(j * LANE, LANE)] = acc
                return carry2

            lax.fori_loop(0, C // LANE, sub, 0)
            pltpu.sync_copy(dbuf, d2_hbm.at[pl.ds(off, C)])
            return carry

        lax.fori_loop(0, steps, chunk, 0)

    return k(pxa, pya, pza, send, rec)


def _sc_gather(T2, U2, send, rec):
    N, D = T2.shape
    E = send.shape[0]
    EW = E // NW
    C = 80
    steps = EW // C

    dt = T2.dtype

    @functools.partial(
        pl.kernel,
        out_type=(jax.ShapeDtypeStruct((E, D), dt),
                  jax.ShapeDtypeStruct((E, D), dt)),
        mesh=_mesh(),
        compiler_params=pltpu.CompilerParams(needs_layout_passes=False),
        scratch_types=[
            pltpu.VMEM((C,), jnp.int32),
            pltpu.VMEM((C,), jnp.int32),
            pltpu.VMEM((C, D), dt),
            pltpu.VMEM((C, D), dt),
            pltpu.SemaphoreType.DMA,
            pltpu.SemaphoreType.DMA,
        ],
    )
    def k(t_hbm, u_hbm, send_hbm, rec_hbm, g1_hbm, g2_hbm,
          sidx, ridx, rows1, rows2, sem1, sem2):
        wid = lax.axis_index("s") * NC + lax.axis_index("c")
        base = wid * EW

        def chunk(i, carry):
            off = base + i * C
            pltpu.sync_copy(send_hbm.at[pl.ds(off, C)], sidx)
            pltpu.sync_copy(rec_hbm.at[pl.ds(off, C)], ridx)
            cp1 = pltpu.async_copy(t_hbm.at[sidx], rows1, sem1)
            cp2 = pltpu.async_copy(u_hbm.at[ridx], rows2, sem2)
            cp1.wait()
            cp2.wait()
            pltpu.sync_copy(rows1, g1_hbm.at[pl.ds(off, C)])
            pltpu.sync_copy(rows2, g2_hbm.at[pl.ds(off, C)])
            return carry

        lax.fori_loop(0, steps, chunk, 0)

    return k(T2, U2, send, rec)


def _sc_scatter(M, rec, zeros):
    _, E, D = M.shape
    NP = zeros.shape[0]
    ET = E // NS
    C = 80
    steps = ET // C
    NT = NP // NS

    @functools.partial(
        pl.kernel,
        out_type=jax.ShapeDtypeStruct((2, NP, D), jnp.float32),
        mesh=_mesh(),
        compiler_params=pltpu.CompilerParams(needs_layout_passes=False),
        scratch_types=[
            pltpu.VMEM((C,), jnp.int32),
            pltpu.VMEM((C, D), jnp.float32),
            pltpu.VMEM_SHARED((NP, D), jnp.float32),
            pltpu.SemaphoreType.DMA,
        ],
    )
    def k(m_hbm, rec_hbm, z_hbm, out_hbm, ridx, rows, acc, sem):
        cid = lax.axis_index("c")
        sid = lax.axis_index("s")
        nb = pl.multiple_of(sid * NT, 8)
        pltpu.sync_copy(z_hbm.at[pl.ds(nb, NT)], acc.at[pl.ds(nb, NT)])
        plsc.subcore_barrier()

        base = sid * ET

        def chunk(i, carry):
            off = base + i * C
            pltpu.sync_copy(rec_hbm.at[pl.ds(off, C)], ridx)
            cp = pltpu.async_copy(m_hbm.at[cid, pl.ds(off, C)], rows, sem)
            cp.wait()
            pltpu.sync_copy(rows, acc.at[ridx], add=True)
            return carry

        lax.fori_loop(0, steps, chunk, 0)
        plsc.subcore_barrier()
        pltpu.sync_copy(acc.at[pl.ds(nb, NT)],
                        out_hbm.at[cid, pl.ds(nb, NT)])

    return k(M, rec, zeros)


_silu = jax.nn.silu


def _tc_embed(x, pe, E1w, E1b, E2w, E2b, G1w, G1b, G2w, G2b):
    N = x.shape[0]
    BN = 2000

    def body(x_ref, pe_ref, e1w, e1b, e2w, e2b, g1w, g1b, g2w, g2b,
             h_ref, peh_ref):
        xin = jnp.concatenate([x_ref[...], pe_ref[...]], axis=1)
        t = _silu(jnp.dot(xin, e1w[...], preferred_element_type=jnp.float32)
                  + e1b[...])
        h_ref[...] = jnp.dot(t, e2w[...], preferred_element_type=jnp.float32) + e2b[...]
        tp = _silu(jnp.dot(pe_ref[...], g1w[...], preferred_element_type=jnp.float32)
                   + g1b[...])
        peh_ref[...] = jnp.dot(tp, g2w[...], preferred_element_type=jnp.float32) + g2b[...]

    full = lambda s: pl.BlockSpec(s, lambda i: (0, 0))
    return pl.pallas_call(
        body,
        grid=(N // BN,),
        in_specs=[
            pl.BlockSpec((BN, x.shape[1]), lambda i: (i, 0)),
            pl.BlockSpec((BN, pe.shape[1]), lambda i: (i, 0)),
            full(E1w.shape), full((1, H)), full(E2w.shape), full((1, H)),
            full(G1w.shape), full((1, H)), full(G2w.shape), full((1, H)),
        ],
        out_specs=[pl.BlockSpec((BN, H), lambda i: (i, 0)),
                   pl.BlockSpec((BN, H), lambda i: (i, 0))],
        out_shape=[jax.ShapeDtypeStruct((N, H), jnp.float32),
                   jax.ShapeDtypeStruct((N, H), jnp.float32)],
    )(x, pe, E1w, E1b[None, :], E2w, E2b[None, :],
      G1w, G1b[None, :], G2w, G2b[None, :])


def _tc_proj(h, pe_h, WT, WU):
    N = h.shape[0]
    BN = 2000

    def _pack(a, b):
        ha = jax.lax.bitcast_convert_type(
            a.astype(jnp.bfloat16).astype(jnp.float32), jnp.uint32)
        hb = jax.lax.bitcast_convert_type(
            b.astype(jnp.bfloat16).astype(jnp.float32), jnp.uint32)
        return ha | (hb >> 16)

    def body(h_ref, pe_ref, wt, wu, t_ref, u_ref):
        z = jnp.concatenate([h_ref[...], pe_ref[...]], axis=1)
        tm = jnp.dot(z, wt[:, :H], preferred_element_type=jnp.float32)
        tp = jnp.dot(z, wt[:, H:], preferred_element_type=jnp.float32)
        um = jnp.dot(z, wu[:, :H], preferred_element_type=jnp.float32)
        up = jnp.dot(z, wu[:, H:], preferred_element_type=jnp.float32)
        t_ref[...] = _pack(tm, tp)
        u_ref[...] = _pack(um, up)

    return pl.pallas_call(
        body,
        grid=(N // BN,),
        in_specs=[
            pl.BlockSpec((BN, H), lambda i: (i, 0)),
            pl.BlockSpec((BN, H), lambda i: (i, 0)),
            pl.BlockSpec(WT.shape, lambda i: (0, 0)),
            pl.BlockSpec(WU.shape, lambda i: (0, 0)),
        ],
        out_specs=[pl.BlockSpec((BN, H), lambda i: (i, 0)),
                   pl.BlockSpec((BN, H), lambda i: (i, 0))],
        out_shape=[jax.ShapeDtypeStruct((N, H), jnp.uint32),
                   jax.ShapeDtypeStruct((N, H), jnp.uint32)],
    )(h, pe_h, WT, WU)


def _tc_edge(g1, g2, d2, vecs, W2, W2p):
    E = g1.shape[0]
    BE = 2000

    def body(g1_ref, g2_ref, d2_ref, v_ref, w2, w2p, m_ref):
        dist = jnp.sqrt(d2_ref[...])
        g1w = g1_ref[...]
        g2w = g2_ref[...]
        hi = jnp.uint32(0xFFFF0000)
        unf = lambda u: jax.lax.bitcast_convert_type(u, jnp.float32)
        gm = unf(g1w & hi) + unf(g2w & hi)
        gp = unf(g1w << 16) + unf(g2w << 16)
        pre1 = gm + dist * v_ref[0:1, :] + v_ref[1:2, :]
        pre1p = gp + dist * v_ref[2:3, :] + v_ref[3:4, :]
        t = _silu(pre1)
        u = jnp.dot(t, w2[...], preferred_element_type=jnp.float32) + v_ref[4:5, :]
        m_ref[0] = _silu(u)
        tp = jnp.tanh(pre1p)
        up = jnp.dot(tp, w2p[...], preferred_element_type=jnp.float32) + v_ref[5:6, :]
        m_ref[1] = jnp.tanh(up)

    return pl.pallas_call(
        body,
        grid=(E // BE,),
        in_specs=[
            pl.BlockSpec((BE, H), lambda i: (i, 0)),
            pl.BlockSpec((BE, H), lambda i: (i, 0)),
            pl.BlockSpec((BE, 1), lambda i: (i, 0)),
            pl.BlockSpec(vecs.shape, lambda i: (0, 0)),
            pl.BlockSpec(W2.shape, lambda i: (0, 0)),
            pl.BlockSpec(W2p.shape, lambda i: (0, 0)),
        ],
        out_specs=pl.BlockSpec((2, BE, H), lambda i: (0, i, 0)),
        out_shape=jax.ShapeDtypeStruct((2, E, H), jnp.float32),
    )(g1, g2, d2, vecs, W2, W2p)


def _tc_update(h, pe_h, aggr, aggrp, V1, c1, V2, c2, P1, p1, P2, p2):
    N = h.shape[0]
    BN = 2000

    def body(h_ref, pe_ref, a_ref, ap_ref, v1, c1r, v2, c2r, q1, p1r, q2, p2r,
             hn_ref, pen_ref):
        cat = jnp.concatenate([h_ref[...], pe_ref[...], a_ref[...]], axis=1)
        z = _silu(jnp.dot(cat, v1[...], preferred_element_type=jnp.float32) + c1r[...])
        upd = jnp.dot(z, v2[...], preferred_element_type=jnp.float32) + c2r[...]
        hn_ref[...] = h_ref[...] + upd
        catp = jnp.concatenate([pe_ref[...], ap_ref[...]], axis=1)
        zp = jnp.tanh(jnp.dot(catp, q1[...], preferred_element_type=jnp.float32) + p1r[...])
        updp = jnp.tanh(jnp.dot(zp, q2[...], preferred_element_type=jnp.float32) + p2r[...])
        pen_ref[...] = pe_ref[...] + updp

    full = lambda s: pl.BlockSpec(s, lambda i: (0, 0))
    return pl.pallas_call(
        body,
        grid=(N // BN,),
        in_specs=[
            pl.BlockSpec((BN, H), lambda i: (i, 0)),
            pl.BlockSpec((BN, H), lambda i: (i, 0)),
            pl.BlockSpec((BN, H), lambda i: (i, 0)),
            pl.BlockSpec((BN, H), lambda i: (i, 0)),
            full(V1.shape), full((1, H)), full(V2.shape), full((1, H)),
            full(P1.shape), full((1, H)), full(P2.shape), full((1, H)),
        ],
        out_specs=[pl.BlockSpec((BN, H), lambda i: (i, 0)),
                   pl.BlockSpec((BN, H), lambda i: (i, 0))],
        out_shape=[jax.ShapeDtypeStruct((N, H), jnp.float32),
                   jax.ShapeDtypeStruct((N, H), jnp.float32)],
    )(h, pe_h, aggr, aggrp, V1, c1[None, :], V2, c2[None, :],
      P1, p1[None, :], P2, p2[None, :])


def _tc_final(h, batch2d, NB, Q1, q1, Q2, q2, R1, r1, R2p, r2p):
    N = h.shape[0]

    def body(h_ref, b_ref, w1, b1r, w2, b2r, w3, b3r, w4, b4r, out_ref):
        t = _silu(jnp.dot(h_ref[...], w1[...], preferred_element_type=jnp.float32)
                  + b1r[...])
        hpre = jnp.dot(t, w2[...], preferred_element_type=jnp.float32) + b2r[...]
        seg = lax.broadcasted_iota(jnp.int32, (NB, N), 0)
        oh = (b_ref[...] == seg).astype(jnp.float32)
        pooled = jnp.dot(oh, hpre, preferred_element_type=jnp.float32)
        tr = _silu(jnp.dot(pooled, w3[...], preferred_element_type=jnp.float32)
                   + b3r[...])
        out_ref[...] = jnp.dot(tr, w4[...], preferred_element_type=jnp.float32) + b4r[...]

    return pl.pallas_call(
        body,
        out_shape=jax.ShapeDtypeStruct((NB, H), jnp.float32),
    )(h, batch2d, Q1, q1[None, :], Q2, q2[None, :],
      R1, r1[None, :], R2p, r2p[None, :])


def kernel(x, pos, pe, params, edge_index, batch):
    N = x.shape[0]
    E = edge_index.shape[1]
    NB = 64

    send = edge_index[0].astype(jnp.int32)
    rec = edge_index[1].astype(jnp.int32)

    d2 = _sc_d2(pos[:, 0], pos[:, 1], pos[:, 2], send, rec)
    d2c = d2[:, None]

    emb = params['embed']
    embp = params['embed_pe']
    h, pe_h = _tc_embed(x, pe,
                        emb[0]['w'], emb[0]['b'], emb[1]['w'], emb[1]['b'],
                        embp[0]['w'], embp[0]['b'], embp[1]['w'], embp[1]['b'])

    NPAD = 10240
    zeros_nh = jnp.zeros((NPAD, H), jnp.float32)

    for lp in params['layers']:
        W1 = lp['message_mlp'][0]['w']
        b1 = lp['message_mlp'][0]['b']
        W2 = lp['message_mlp'][1]['w']
        b2 = lp['message_mlp'][1]['b']
        Wp1 = lp['message_pos_mlp'][0]['w']
        bp1 = lp['message_pos_mlp'][0]['b']
        Wp2 = lp['message_pos_mlp'][1]['w']
        bp2 = lp['message_pos_mlp'][1]['b']

        zblk = jnp.zeros((H, H), jnp.float32)
        WT = jnp.concatenate([
            jnp.concatenate([W1[:H], zblk], axis=1),
            jnp.concatenate([W1[H:2 * H], Wp1[:H]], axis=1)], axis=0)
        WU = jnp.concatenate([
            jnp.concatenate([W1[2 * H:3 * H], zblk], axis=1),
            jnp.concatenate([W1[3 * H:4 * H], Wp1[H:2 * H]], axis=1)], axis=0)
        vecs = jnp.stack([W1[4 * H], b1, Wp1[2 * H], bp1, b2, bp2,
                          jnp.zeros((H,), jnp.float32),
                          jnp.zeros((H,), jnp.float32)], axis=0)

        T2, U2 = _tc_proj(h, pe_h, WT, WU)
        g1, g2 = _sc_gather(T2, U2, send, rec)
        M = _tc_edge(g1, g2, d2c, vecs, W2, Wp2)
        A = _sc_scatter(M, rec, zeros_nh)

        h, pe_h = _tc_update(h, pe_h, A[0, :N], A[1, :N],
                             lp['update_mlp'][0]['w'], lp['update_mlp'][0]['b'],
                             lp['update_mlp'][1]['w'], lp['update_mlp'][1]['b'],
                             lp['update_pos_mlp'][0]['w'], lp['update_pos_mlp'][0]['b'],
                             lp['update_pos_mlp'][1]['w'], lp['update_pos_mlp'][1]['b'])

    pr = params['pre_readout']
    ro = params['readout']
    R2 = ro[1]['w']
    R2p = jnp.concatenate([R2, jnp.zeros((H, H - 1), jnp.float32)], axis=1)
    r2p = jnp.concatenate([ro[1]['b'], jnp.zeros((H - 1,), jnp.float32)], axis=0)
    batch2d = batch.astype(jnp.int32)[None, :]
    out = _tc_final(h, batch2d, NB,
                    pr[0]['w'], pr[0]['b'], pr[1]['w'], pr[1]['b'],
                    ro[0]['w'], ro[0]['b'], R2p, r2p)
    return out[:, 0]


# speedup vs baseline: 5.2249x; 1.5869x over previous
import functools

import jax
import jax.numpy as jnp
from jax import lax
from jax.experimental import pallas as pl
from jax.experimental.pallas import tpu as pltpu
from jax.experimental.pallas import tpu_sc as plsc

NC = 2
NS = 16
NW = NC * NS
LANE = 16

H = 128


def _mesh():
    return plsc.VectorSubcoreMesh(core_axis_name="c", subcore_axis_name="s",
                                  num_cores=NC, num_subcores=NS)


def _sc_d2(pxa, pya, pza, send, rec):
    N = pxa.shape[0]
    E = send.shape[0]
    EW = E // NW
    C = 80
    steps = EW // C

    @functools.partial(
        pl.kernel,
        out_type=jax.ShapeDtypeStruct((E,), jnp.float32),
        mesh=_mesh(),
        compiler_params=pltpu.CompilerParams(needs_layout_passes=False),
        scratch_types=[
            pltpu.VMEM((N,), jnp.float32),
            pltpu.VMEM((N,), jnp.float32),
            pltpu.VMEM((N,), jnp.float32),
            pltpu.VMEM((C,), jnp.int32),
            pltpu.VMEM((C,), jnp.int32),
            pltpu.VMEM((C,), jnp.float32),
        ],
    )
    def k(px_hbm, py_hbm, pz_hbm, send_hbm, rec_hbm, d2_hbm,
          px, py, pz, sidx, ridx, dbuf):
        wid = lax.axis_index("s") * NC + lax.axis_index("c")
        base = wid * EW
        pltpu.sync_copy(px_hbm, px)
        pltpu.sync_copy(py_hbm, py)
        pltpu.sync_copy(pz_hbm, pz)

        def chunk(i, carry):
            off = base + i * C
            pltpu.sync_copy(send_hbm.at[pl.ds(off, C)], sidx)
            pltpu.sync_copy(rec_hbm.at[pl.ds(off, C)], ridx)

            def sub(j, carry2):
                s16 = sidx[pl.ds(j * LANE, LANE)]
                r16 = ridx[pl.ds(j * LANE, LANE)]
                acc = jnp.zeros((LANE,), jnp.float32)
                for tab in (px, py, pz):
                    a = plsc.load_gather(tab, [s16])
                    b = plsc.load_gather(tab, [r16])
                    d = a - b
                    acc = acc + d * d
                dbuf[pl.ds(j * LANE, LANE)] = acc
                return carry2

            lax.fori_loop(0, C // LANE, sub, 0)
            pltpu.sync_copy(dbuf, d2_hbm.at[pl.ds(off, C)])
            return carry

        lax.fori_loop(0, steps, chunk, 0)

    return k(pxa, pya, pza, send, rec)


def _sc_gather(T2, U2, send, rec):
    N, D = T2.shape
    E = send.shape[0]
    EW = E // NW
    C = 80
    steps = EW // C
    assert C % 8 == 0 and EW % C == 0 and steps % 2 == 1 and steps >= 3
    pairs = steps // 2

    dt = T2.dtype

    @functools.partial(
        pl.kernel,
        out_type=(jax.ShapeDtypeStruct((E, D), dt),
                  jax.ShapeDtypeStruct((E, D), dt)),
        mesh=_mesh(),
        compiler_params=pltpu.CompilerParams(needs_layout_passes=False),
        scratch_types=[
            pltpu.VMEM((C,), jnp.int32),
            pltpu.VMEM((C,), jnp.int32),
            pltpu.VMEM((C,), jnp.int32),
            pltpu.VMEM((C,), jnp.int32),
            pltpu.VMEM((C, D), dt),
            pltpu.VMEM((C, D), dt),
            pltpu.VMEM((C, D), dt),
            pltpu.VMEM((C, D), dt),
            pltpu.SemaphoreType.DMA,
            pltpu.SemaphoreType.DMA,
            pltpu.SemaphoreType.DMA,
            pltpu.SemaphoreType.DMA,
            pltpu.SemaphoreType.DMA,
            pltpu.SemaphoreType.DMA,
        ],
    )
    def k(t_hbm, u_hbm, send_hbm, rec_hbm, g1_hbm, g2_hbm,
          sidx0, sidx1, ridx0, ridx1, r10, r11, r20, r21,
          isem0, isem1, gsem0, gsem1, wsem0, wsem1):
        wid = lax.axis_index("s") * NC + lax.axis_index("c")
        base = wid * EW
        sidx = (sidx0, sidx1)
        ridx = (ridx0, ridx1)
        r1 = (r10, r11)
        r2 = (r20, r21)
        isem = (isem0, isem1)
        gsem = (gsem0, gsem1)
        wsem = (wsem0, wsem1)

        def idx_issue(i, b):
            off = base + i * C
            pltpu.async_copy(send_hbm.at[pl.ds(off, C)], sidx[b], isem[b])
            pltpu.async_copy(rec_hbm.at[pl.ds(off, C)], ridx[b], isem[b])

        def idx_wait(b):
            pltpu.make_async_copy(send_hbm.at[pl.ds(0, C)], sidx[b], isem[b]).wait()
            pltpu.make_async_copy(rec_hbm.at[pl.ds(0, C)], ridx[b], isem[b]).wait()

        def gat_issue(b):
            pltpu.async_copy(t_hbm.at[sidx[b]], r1[b], gsem[b])
            pltpu.async_copy(u_hbm.at[ridx[b]], r2[b], gsem[b])

        def gat_wait(b):
            pltpu.make_async_copy(t_hbm.at[pl.ds(0, C)], r1[b], gsem[b]).wait()
            pltpu.make_async_copy(u_hbm.at[pl.ds(0, C)], r2[b], gsem[b]).wait()

        def wr_issue(i, b):
            off = base + i * C
            pltpu.async_copy(r1[b], g1_hbm.at[pl.ds(off, C)], wsem[b])
            pltpu.async_copy(r2[b], g2_hbm.at[pl.ds(off, C)], wsem[b])

        def wr_wait(b):
            pltpu.make_async_copy(r1[b], g1_hbm.at[pl.ds(0, C)], wsem[b]).wait()
            pltpu.make_async_copy(r2[b], g2_hbm.at[pl.ds(0, C)], wsem[b]).wait()

        idx_issue(0, 0)
        idx_issue(1, 1)
        idx_wait(0)
        gat_issue(0)

        def pair(kk, carry):
            i0 = 2 * kk

            @pl.when(kk > 0)
            def _():
                wr_wait(1)
            idx_wait(1)
            gat_issue(1)
            gat_wait(0)
            wr_issue(i0, 0)
            idx_issue(i0 + 2, 0)
            wr_wait(0)
            idx_wait(0)
            gat_issue(0)
            gat_wait(1)
            wr_issue(i0 + 1, 1)

            @pl.when(kk < pairs - 1)
            def _():
                idx_issue(i0 + 3, 1)
            return carry

        lax.fori_loop(0, pairs, pair, 0)
        gat_wait(0)
        wr_wait(1)
        wr_issue(steps - 1, 0)
        wr_wait(0)

    return k(T2, U2, send, rec)


def _sc_scatter(M, rec, zeros):
    _, E, D = M.shape
    NP = zeros.shape[0]
    ET = E // NS
    C = 80
    steps = ET // C
    NT = NP // NS
    assert steps % 4 == 2 and steps >= 6
    quads = (steps - 2) // 4

    @functools.partial(
        pl.kernel,
        out_type=jax.ShapeDtypeStruct((2, NP, D), jnp.float32),
        mesh=_mesh(),
        compiler_params=pltpu.CompilerParams(needs_layout_passes=False),
        scratch_types=[
            pltpu.VMEM((C,), jnp.int32),
            pltpu.VMEM((C,), jnp.int32),
            pltpu.VMEM((C,), jnp.int32),
            pltpu.VMEM((C,), jnp.int32),
            pltpu.VMEM((C, D), jnp.float32),
            pltpu.VMEM((C, D), jnp.float32),
            pltpu.VMEM((C, D), jnp.float32),
            pltpu.VMEM((C, D), jnp.float32),
            pltpu.VMEM_SHARED((NP, D), jnp.float32),
            pltpu.SemaphoreType.DMA,
            pltpu.SemaphoreType.DMA,
            pltpu.SemaphoreType.DMA,
            pltpu.SemaphoreType.DMA,
            pltpu.SemaphoreType.DMA,
            pltpu.SemaphoreType.DMA,
            pltpu.SemaphoreType.DMA,
            pltpu.SemaphoreType.DMA,
        ],
    )
    def k(m_hbm, rec_hbm, z_hbm, out_hbm,
          ridx0, ridx1, ridx2, ridx3, rows0, rows1, rows2, rows3,
          acc, lsem0, lsem1, lsem2, lsem3, ssem0, ssem1, ssem2, ssem3):
        cid = lax.axis_index("c")
        sid = lax.axis_index("s")
        nb = pl.multiple_of(sid * NT, 8)
        base = sid * ET
        ridx = (ridx0, ridx1, ridx2, ridx3)
        rows = (rows0, rows1, rows2, rows3)
        lsem = (lsem0, lsem1, lsem2, lsem3)
        ssem = (ssem0, ssem1, ssem2, ssem3)

        def ld_issue(i, b):
            off = base + i * C
            pltpu.async_copy(rec_hbm.at[pl.ds(off, C)], ridx[b], lsem[b])
            pltpu.async_copy(m_hbm.at[cid, pl.ds(off, C)], rows[b], lsem[b])

        def ld_wait(b):
            pltpu.make_async_copy(rec_hbm.at[pl.ds(0, C)], ridx[b], lsem[b]).wait()
            pltpu.make_async_copy(m_hbm.at[cid, pl.ds(0, C)], rows[b], lsem[b]).wait()

        def scat_issue(b):
            pltpu.async_copy(rows[b], acc.at[ridx[b]], ssem[b], add=True)

        def scat_wait(b):
            pltpu.make_async_copy(rows[b], acc.at[pl.ds(0, C)], ssem[b]).wait()

        ld_issue(0, 0)
        pltpu.sync_copy(z_hbm.at[pl.ds(nb, NT)], acc.at[pl.ds(nb, NT)])
        plsc.subcore_barrier()
        ld_issue(1, 1)
        ld_issue(2, 2)

        def step(j, b, kk, first_quad, last_quad):
            ld_wait(b)
            scat_issue(b)
            prev = (b - 1) % 4
            if first_quad is None:
                scat_wait(prev)
            else:
                @pl.when(kk > 0)
                def _():
                    scat_wait(prev)
            nxt = (b + 3) % 4
            if last_quad is None:
                ld_issue(j + 3, nxt)
            elif last_quad:
                @pl.when(kk < quads - 1)
                def _():
                    ld_issue(j + 3, nxt)

        def quad(kk, carry):
            j0 = 4 * kk
            step(j0 + 0, 0, kk, True, None)
            step(j0 + 1, 1, kk, None, None)
            step(j0 + 2, 2, kk, None, None)
            step(j0 + 3, 3, kk, None, True)
            return carry

        lax.fori_loop(0, quads, quad, 0)
        ld_wait(0)
        scat_issue(0)
        scat_wait(3)
        ld_wait(1)
        scat_issue(1)
        scat_wait(0)
        scat_wait(1)
        plsc.subcore_barrier()
        pltpu.sync_copy(acc.at[pl.ds(nb, NT)],
                        out_hbm.at[cid, pl.ds(nb, NT)])

    return k(M, rec, zeros)


_silu = jax.nn.silu


def _tc_embed(x, pe, E1w, E1b, E2w, E2b, G1w, G1b, G2w, G2b):
    N = x.shape[0]
    BN = 2000

    def body(x_ref, pe_ref, e1w, e1b, e2w, e2b, g1w, g1b, g2w, g2b,
             h_ref, peh_ref):
        xin = jnp.concatenate([x_ref[...], pe_ref[...]], axis=1)
        t = _silu(jnp.dot(xin, e1w[...], preferred_element_type=jnp.float32)
                  + e1b[...])
        h_ref[...] = jnp.dot(t, e2w[...], preferred_element_type=jnp.float32) + e2b[...]
        tp = _silu(jnp.dot(pe_ref[...], g1w[...], preferred_element_type=jnp.float32)
                   + g1b[...])
        peh_ref[...] = jnp.dot(tp, g2w[...], preferred_element_type=jnp.float32) + g2b[...]

    full = lambda s: pl.BlockSpec(s, lambda i: (0, 0))
    return pl.pallas_call(
        body,
        grid=(N // BN,),
        in_specs=[
            pl.BlockSpec((BN, x.shape[1]), lambda i: (i, 0)),
            pl.BlockSpec((BN, pe.shape[1]), lambda i: (i, 0)),
            full(E1w.shape), full((1, H)), full(E2w.shape), full((1, H)),
            full(G1w.shape), full((1, H)), full(G2w.shape), full((1, H)),
        ],
        out_specs=[pl.BlockSpec((BN, H), lambda i: (i, 0)),
                   pl.BlockSpec((BN, H), lambda i: (i, 0))],
        out_shape=[jax.ShapeDtypeStruct((N, H), jnp.float32),
                   jax.ShapeDtypeStruct((N, H), jnp.float32)],
    )(x, pe, E1w, E1b[None, :], E2w, E2b[None, :],
      G1w, G1b[None, :], G2w, G2b[None, :])


def _tc_proj(h, pe_h, WT, WU):
    N = h.shape[0]
    BN = 2000

    def _pack(a, b):
        ha = jax.lax.bitcast_convert_type(
            a.astype(jnp.bfloat16).astype(jnp.float32), jnp.uint32)
        hb = jax.lax.bitcast_convert_type(
            b.astype(jnp.bfloat16).astype(jnp.float32), jnp.uint32)
        return ha | (hb >> 16)

    def body(h_ref, pe_ref, wt, wu, t_ref, u_ref):
        z = jnp.concatenate([h_ref[...], pe_ref[...]], axis=1)
        tm = jnp.dot(z, wt[:, :H], preferred_element_type=jnp.float32)
        tp = jnp.dot(z, wt[:, H:], preferred_element_type=jnp.float32)
        um = jnp.dot(z, wu[:, :H], preferred_element_type=jnp.float32)
        up = jnp.dot(z, wu[:, H:], preferred_element_type=jnp.float32)
        t_ref[...] = _pack(tm, tp)
        u_ref[...] = _pack(um, up)

    return pl.pallas_call(
        body,
        grid=(N // BN,),
        in_specs=[
            pl.BlockSpec((BN, H), lambda i: (i, 0)),
            pl.BlockSpec((BN, H), lambda i: (i, 0)),
            pl.BlockSpec(WT.shape, lambda i: (0, 0)),
            pl.BlockSpec(WU.shape, lambda i: (0, 0)),
        ],
        out_specs=[pl.BlockSpec((BN, H), lambda i: (i, 0)),
                   pl.BlockSpec((BN, H), lambda i: (i, 0))],
        out_shape=[jax.ShapeDtypeStruct((N, H), jnp.uint32),
                   jax.ShapeDtypeStruct((N, H), jnp.uint32)],
    )(h, pe_h, WT, WU)


def _tc_edge(g1, g2, d2, vecs, W2, W2p):
    E = g1.shape[0]
    BE = 2000

    def body(g1_ref, g2_ref, d2_ref, v_ref, w2, w2p, m_ref):
        dist = jnp.sqrt(d2_ref[...])
        g1w = g1_ref[...]
        g2w = g2_ref[...]
        hi = jnp.uint32(0xFFFF0000)
        unf = lambda u: jax.lax.bitcast_convert_type(u, jnp.float32)
        gm = unf(g1w & hi) + unf(g2w & hi)
        gp = unf(g1w << 16) + unf(g2w << 16)
        pre1 = gm + dist * v_ref[0:1, :] + v_ref[1:2, :]
        pre1p = gp + dist * v_ref[2:3, :] + v_ref[3:4, :]
        t = _silu(pre1)
        u = jnp.dot(t, w2[...], preferred_element_type=jnp.float32) + v_ref[4:5, :]
        m_ref[0] = _silu(u)
        tp = jnp.tanh(pre1p)
        up = jnp.dot(tp, w2p[...], preferred_element_type=jnp.float32) + v_ref[5:6, :]
        m_ref[1] = jnp.tanh(up)

    return pl.pallas_call(
        body,
        grid=(E // BE,),
        in_specs=[
            pl.BlockSpec((BE, H), lambda i: (i, 0)),
            pl.BlockSpec((BE, H), lambda i: (i, 0)),
            pl.BlockSpec((BE, 1), lambda i: (i, 0)),
            pl.BlockSpec(vecs.shape, lambda i: (0, 0)),
            pl.BlockSpec(W2.shape, lambda i: (0, 0)),
            pl.BlockSpec(W2p.shape, lambda i: (0, 0)),
        ],
        out_specs=pl.BlockSpec((2, BE, H), lambda i: (0, i, 0)),
        out_shape=jax.ShapeDtypeStruct((2, E, H), jnp.float32),
    )(g1, g2, d2, vecs, W2, W2p)


def _tc_update(h, pe_h, aggr, aggrp, V1, c1, V2, c2, P1, p1, P2, p2):
    N = h.shape[0]
    BN = 2000

    def body(h_ref, pe_ref, a_ref, ap_ref, v1, c1r, v2, c2r, q1, p1r, q2, p2r,
             hn_ref, pen_ref):
        cat = jnp.concatenate([h_ref[...], pe_ref[...], a_ref[...]], axis=1)
        z = _silu(jnp.dot(cat, v1[...], preferred_element_type=jnp.float32) + c1r[...])
        upd = jnp.dot(z, v2[...], preferred_element_type=jnp.float32) + c2r[...]
        hn_ref[...] = h_ref[...] + upd
        catp = jnp.concatenate([pe_ref[...], ap_ref[...]], axis=1)
        zp = jnp.tanh(jnp.dot(catp, q1[...], preferred_element_type=jnp.float32) + p1r[...])
        updp = jnp.tanh(jnp.dot(zp, q2[...], preferred_element_type=jnp.float32) + p2r[...])
        pen_ref[...] = pe_ref[...] + updp

    full = lambda s: pl.BlockSpec(s, lambda i: (0, 0))
    return pl.pallas_call(
        body,
        grid=(N // BN,),
        in_specs=[
            pl.BlockSpec((BN, H), lambda i: (i, 0)),
            pl.BlockSpec((BN, H), lambda i: (i, 0)),
            pl.BlockSpec((BN, H), lambda i: (i, 0)),
            pl.BlockSpec((BN, H), lambda i: (i, 0)),
            full(V1.shape), full((1, H)), full(V2.shape), full((1, H)),
            full(P1.shape), full((1, H)), full(P2.shape), full((1, H)),
        ],
        out_specs=[pl.BlockSpec((BN, H), lambda i: (i, 0)),
                   pl.BlockSpec((BN, H), lambda i: (i, 0))],
        out_shape=[jax.ShapeDtypeStruct((N, H), jnp.float32),
                   jax.ShapeDtypeStruct((N, H), jnp.float32)],
    )(h, pe_h, aggr, aggrp, V1, c1[None, :], V2, c2[None, :],
      P1, p1[None, :], P2, p2[None, :])


def _tc_final(h, batch2d, NB, Q1, q1, Q2, q2, R1, r1, R2p, r2p):
    N = h.shape[0]

    def body(h_ref, b_ref, w1, b1r, w2, b2r, w3, b3r, w4, b4r, out_ref):
        t = _silu(jnp.dot(h_ref[...], w1[...], preferred_element_type=jnp.float32)
                  + b1r[...])
        hpre = jnp.dot(t, w2[...], preferred_element_type=jnp.float32) + b2r[...]
        seg = lax.broadcasted_iota(jnp.int32, (NB, N), 0)
        oh = (b_ref[...] == seg).astype(jnp.float32)
        pooled = jnp.dot(oh, hpre, preferred_element_type=jnp.float32)
        tr = _silu(jnp.dot(pooled, w3[...], preferred_element_type=jnp.float32)
                   + b3r[...])
        out_ref[...] = jnp.dot(tr, w4[...], preferred_element_type=jnp.float32) + b4r[...]

    return pl.pallas_call(
        body,
        out_shape=jax.ShapeDtypeStruct((NB, H), jnp.float32),
    )(h, batch2d, Q1, q1[None, :], Q2, q2[None, :],
      R1, r1[None, :], R2p, r2p[None, :])


def kernel(x, pos, pe, params, edge_index, batch):
    N = x.shape[0]
    E = edge_index.shape[1]
    NB = 64

    send = edge_index[0].astype(jnp.int32)
    rec = edge_index[1].astype(jnp.int32)

    d2 = _sc_d2(pos[:, 0], pos[:, 1], pos[:, 2], send, rec)
    d2c = d2[:, None]

    emb = params['embed']
    embp = params['embed_pe']
    h, pe_h = _tc_embed(x, pe,
                        emb[0]['w'], emb[0]['b'], emb[1]['w'], emb[1]['b'],
                        embp[0]['w'], embp[0]['b'], embp[1]['w'], embp[1]['b'])

    NPAD = 10240
    zeros_nh = jnp.zeros((NPAD, H), jnp.float32)

    for lp in params['layers']:
        W1 = lp['message_mlp'][0]['w']
        b1 = lp['message_mlp'][0]['b']
        W2 = lp['message_mlp'][1]['w']
        b2 = lp['message_mlp'][1]['b']
        Wp1 = lp['message_pos_mlp'][0]['w']
        bp1 = lp['message_pos_mlp'][0]['b']
        Wp2 = lp['message_pos_mlp'][1]['w']
        bp2 = lp['message_pos_mlp'][1]['b']

        zblk = jnp.zeros((H, H), jnp.float32)
        WT = jnp.concatenate([
            jnp.concatenate([W1[:H], zblk], axis=1),
            jnp.concatenate([W1[H:2 * H], Wp1[:H]], axis=1)], axis=0)
        WU = jnp.concatenate([
            jnp.concatenate([W1[2 * H:3 * H], zblk], axis=1),
            jnp.concatenate([W1[3 * H:4 * H], Wp1[H:2 * H]], axis=1)], axis=0)
        vecs = jnp.stack([W1[4 * H], b1, Wp1[2 * H], bp1, b2, bp2,
                          jnp.zeros((H,), jnp.float32),
                          jnp.zeros((H,), jnp.float32)], axis=0)

        T2, U2 = _tc_proj(h, pe_h, WT, WU)
        g1, g2 = _sc_gather(T2, U2, send, rec)
        M = _tc_edge(g1, g2, d2c, vecs, W2, Wp2)
        A = _sc_scatter(M, rec, zeros_nh)

        h, pe_h = _tc_update(h, pe_h, A[0, :N], A[1, :N],
                             lp['update_mlp'][0]['w'], lp['update_mlp'][0]['b'],
                             lp['update_mlp'][1]['w'], lp['update_mlp'][1]['b'],
                             lp['update_pos_mlp'][0]['w'], lp['update_pos_mlp'][0]['b'],
                             lp['update_pos_mlp'][1]['w'], lp['update_pos_mlp'][1]['b'])

    pr = params['pre_readout']
    ro = params['readout']
    R2 = ro[1]['w']
    R2p = jnp.concatenate([R2, jnp.zeros((H, H - 1), jnp.float32)], axis=1)
    r2p = jnp.concatenate([ro[1]['b'], jnp.zeros((H - 1,), jnp.float32)], axis=0)
    batch2d = batch.astype(jnp.int32)[None, :]
    out = _tc_final(h, batch2d, NB,
                    pr[0]['w'], pr[0]['b'], pr[1]['w'], pr[1]['b'],
                    ro[0]['w'], ro[0]['b'], R2p, r2p)
    return out[:, 0]


# speedup vs baseline: 5.3670x; 1.0272x over previous
import functools

import jax
import jax.numpy as jnp
from jax import lax
from jax.experimental import pallas as pl
from jax.experimental.pallas import tpu as pltpu
from jax.experimental.pallas import tpu_sc as plsc

NC = 2
NS = 16
NW = NC * NS
LANE = 16

H = 128


def _mesh():
    return plsc.VectorSubcoreMesh(core_axis_name="c", subcore_axis_name="s",
                                  num_cores=NC, num_subcores=NS)


def _sc_d2(pxa, pya, pza, send, rec):
    N = pxa.shape[0]
    E = send.shape[0]
    EW = E // NW
    C = 80
    steps = EW // C

    @functools.partial(
        pl.kernel,
        out_type=jax.ShapeDtypeStruct((E,), jnp.float32),
        mesh=_mesh(),
        compiler_params=pltpu.CompilerParams(needs_layout_passes=False),
        scratch_types=[
            pltpu.VMEM((N,), jnp.float32),
            pltpu.VMEM((N,), jnp.float32),
            pltpu.VMEM((N,), jnp.float32),
            pltpu.VMEM((C,), jnp.int32),
            pltpu.VMEM((C,), jnp.int32),
            pltpu.VMEM((C,), jnp.float32),
        ],
    )
    def k(px_hbm, py_hbm, pz_hbm, send_hbm, rec_hbm, d2_hbm,
          px, py, pz, sidx, ridx, dbuf):
        wid = lax.axis_index("s") * NC + lax.axis_index("c")
        base = wid * EW
        pltpu.sync_copy(px_hbm, px)
        pltpu.sync_copy(py_hbm, py)
        pltpu.sync_copy(pz_hbm, pz)

        def chunk(i, carry):
            off = base + i * C
            pltpu.sync_copy(send_hbm.at[pl.ds(off, C)], sidx)
            pltpu.sync_copy(rec_hbm.at[pl.ds(off, C)], ridx)

            def sub(j, carry2):
                s16 = sidx[pl.ds(j * LANE, LANE)]
                r16 = ridx[pl.ds(j * LANE, LANE)]
                acc = jnp.zeros((LANE,), jnp.float32)
                for tab in (px, py, pz):
                    a = plsc.load_gather(tab, [s16])
                    b = plsc.load_gather(tab, [r16])
                    d = a - b
                    acc = acc + d * d
                dbuf[pl.ds(j * LANE, LANE)] = acc
                return carry2

            lax.fori_loop(0, C // LANE, sub, 0)
            pltpu.sync_copy(dbuf, d2_hbm.at[pl.ds(off, C)])
            return carry

        lax.fori_loop(0, steps, chunk, 0)

    return k(pxa, pya, pza, send, rec)


def _sc_gather(T2, U2, send, rec):
    N, D = T2.shape
    E = send.shape[0]
    EW = E // NW
    C = 80
    steps = EW // C
    assert C % 8 == 0 and EW % C == 0 and steps % 2 == 1 and steps >= 3
    pairs = steps // 2

    dt = T2.dtype

    @functools.partial(
        pl.kernel,
        out_type=(jax.ShapeDtypeStruct((E, D), dt),
                  jax.ShapeDtypeStruct((E, D), dt)),
        mesh=_mesh(),
        compiler_params=pltpu.CompilerParams(needs_layout_passes=False),
        scratch_types=[
            pltpu.VMEM((C,), jnp.int32),
            pltpu.VMEM((C,), jnp.int32),
            pltpu.VMEM((C,), jnp.int32),
            pltpu.VMEM((C,), jnp.int32),
            pltpu.VMEM((C, D), dt),
            pltpu.VMEM((C, D), dt),
            pltpu.VMEM((C, D), dt),
            pltpu.VMEM((C, D), dt),
            pltpu.SemaphoreType.DMA,
            pltpu.SemaphoreType.DMA,
            pltpu.SemaphoreType.DMA,
            pltpu.SemaphoreType.DMA,
            pltpu.SemaphoreType.DMA,
            pltpu.SemaphoreType.DMA,
        ],
    )
    def k(t_hbm, u_hbm, send_hbm, rec_hbm, g1_hbm, g2_hbm,
          sidx0, sidx1, ridx0, ridx1, r10, r11, r20, r21,
          isem0, isem1, gsem0, gsem1, wsem0, wsem1):
        wid = lax.axis_index("s") * NC + lax.axis_index("c")
        base = wid * EW
        sidx = (sidx0, sidx1)
        ridx = (ridx0, ridx1)
        r1 = (r10, r11)
        r2 = (r20, r21)
        isem = (isem0, isem1)
        gsem = (gsem0, gsem1)
        wsem = (wsem0, wsem1)

        def idx_issue(i, b):
            off = base + i * C
            pltpu.async_copy(send_hbm.at[pl.ds(off, C)], sidx[b], isem[b])
            pltpu.async_copy(rec_hbm.at[pl.ds(off, C)], ridx[b], isem[b])

        def idx_wait(b):
            pltpu.make_async_copy(send_hbm.at[pl.ds(0, C)], sidx[b], isem[b]).wait()
            pltpu.make_async_copy(rec_hbm.at[pl.ds(0, C)], ridx[b], isem[b]).wait()

        def gat_issue(b):
            pltpu.async_copy(t_hbm.at[sidx[b]], r1[b], gsem[b])
            pltpu.async_copy(u_hbm.at[ridx[b]], r2[b], gsem[b])

        def gat_wait(b):
            pltpu.make_async_copy(t_hbm.at[pl.ds(0, C)], r1[b], gsem[b]).wait()
            pltpu.make_async_copy(u_hbm.at[pl.ds(0, C)], r2[b], gsem[b]).wait()

        def wr_issue(i, b):
            off = base + i * C
            pltpu.async_copy(r1[b], g1_hbm.at[pl.ds(off, C)], wsem[b])
            pltpu.async_copy(r2[b], g2_hbm.at[pl.ds(off, C)], wsem[b])

        def wr_wait(b):
            pltpu.make_async_copy(r1[b], g1_hbm.at[pl.ds(0, C)], wsem[b]).wait()
            pltpu.make_async_copy(r2[b], g2_hbm.at[pl.ds(0, C)], wsem[b]).wait()

        idx_issue(0, 0)
        idx_issue(1, 1)
        idx_wait(0)
        gat_issue(0)

        def pair(kk, carry):
            i0 = 2 * kk

            @pl.when(kk > 0)
            def _():
                wr_wait(1)
            idx_wait(1)
            gat_issue(1)
            gat_wait(0)
            wr_issue(i0, 0)
            idx_issue(i0 + 2, 0)
            wr_wait(0)
            idx_wait(0)
            gat_issue(0)
            gat_wait(1)
            wr_issue(i0 + 1, 1)

            @pl.when(kk < pairs - 1)
            def _():
                idx_issue(i0 + 3, 1)
            return carry

        lax.fori_loop(0, pairs, pair, 0)
        gat_wait(0)
        wr_wait(1)
        wr_issue(steps - 1, 0)
        wr_wait(0)

    return k(T2, U2, send, rec)


def _sc_scatter(M, rec, zeros):
    _, E, D = M.shape
    NP = zeros.shape[0]
    ET = E // NS
    C = 80
    steps = ET // C
    NT = NP // NS
    assert steps % 4 == 2 and steps >= 6
    quads = (steps - 2) // 4

    @functools.partial(
        pl.kernel,
        out_type=jax.ShapeDtypeStruct((2, NP, D), jnp.float32),
        mesh=_mesh(),
        compiler_params=pltpu.CompilerParams(needs_layout_passes=False),
        scratch_types=[
            pltpu.VMEM((C,), jnp.int32),
            pltpu.VMEM((C,), jnp.int32),
            pltpu.VMEM((C,), jnp.int32),
            pltpu.VMEM((C,), jnp.int32),
            pltpu.VMEM((C, D), jnp.float32),
            pltpu.VMEM((C, D), jnp.float32),
            pltpu.VMEM((C, D), jnp.float32),
            pltpu.VMEM((C, D), jnp.float32),
            pltpu.VMEM_SHARED((NP, D), jnp.float32),
            pltpu.SemaphoreType.DMA,
            pltpu.SemaphoreType.DMA,
            pltpu.SemaphoreType.DMA,
            pltpu.SemaphoreType.DMA,
            pltpu.SemaphoreType.DMA,
            pltpu.SemaphoreType.DMA,
            pltpu.SemaphoreType.DMA,
            pltpu.SemaphoreType.DMA,
        ],
    )
    def k(m_hbm, rec_hbm, z_hbm, out_hbm,
          ridx0, ridx1, ridx2, ridx3, rows0, rows1, rows2, rows3,
          acc, lsem0, lsem1, lsem2, lsem3, ssem0, ssem1, ssem2, ssem3):
        cid = lax.axis_index("c")
        sid = lax.axis_index("s")
        nb = pl.multiple_of(sid * NT, 8)
        base = sid * ET
        ridx = (ridx0, ridx1, ridx2, ridx3)
        rows = (rows0, rows1, rows2, rows3)
        lsem = (lsem0, lsem1, lsem2, lsem3)
        ssem = (ssem0, ssem1, ssem2, ssem3)

        def ld_issue(i, b):
            off = base + i * C
            pltpu.async_copy(rec_hbm.at[pl.ds(off, C)], ridx[b], lsem[b])
            pltpu.async_copy(m_hbm.at[cid, pl.ds(off, C)], rows[b], lsem[b])

        def ld_wait(b):
            pltpu.make_async_copy(rec_hbm.at[pl.ds(0, C)], ridx[b], lsem[b]).wait()
            pltpu.make_async_copy(m_hbm.at[cid, pl.ds(0, C)], rows[b], lsem[b]).wait()

        def scat_issue(b):
            pltpu.async_copy(rows[b], acc.at[ridx[b]], ssem[b], add=True)

        def scat_wait(b):
            pltpu.make_async_copy(rows[b], acc.at[pl.ds(0, C)], ssem[b]).wait()

        ld_issue(0, 0)
        pltpu.sync_copy(z_hbm.at[pl.ds(nb, NT)], acc.at[pl.ds(nb, NT)])
        plsc.subcore_barrier()
        ld_issue(1, 1)
        ld_issue(2, 2)

        def step(j, b, kk, first_quad, last_quad):
            ld_wait(b)
            scat_issue(b)
            prev = (b - 1) % 4
            if first_quad is None:
                scat_wait(prev)
            else:
                @pl.when(kk > 0)
                def _():
                    scat_wait(prev)
            nxt = (b + 3) % 4
            if last_quad is None:
                ld_issue(j + 3, nxt)
            elif last_quad:
                @pl.when(kk < quads - 1)
                def _():
                    ld_issue(j + 3, nxt)

        def quad(kk, carry):
            j0 = 4 * kk
            step(j0 + 0, 0, kk, True, None)
            step(j0 + 1, 1, kk, None, None)
            step(j0 + 2, 2, kk, None, None)
            step(j0 + 3, 3, kk, None, True)
            return carry

        lax.fori_loop(0, quads, quad, 0)
        ld_wait(0)
        scat_issue(0)
        scat_wait(3)
        ld_wait(1)
        scat_issue(1)
        scat_wait(0)
        scat_wait(1)
        plsc.subcore_barrier()
        pltpu.sync_copy(acc.at[pl.ds(nb, NT)],
                        out_hbm.at[cid, pl.ds(nb, NT)])

    return k(M, rec, zeros)


_silu = jax.nn.silu


def _tc_dist(d2r):
    def body(d_ref, o_ref):
        o_ref[...] = jnp.sqrt(d_ref[...])

    return pl.pallas_call(
        body,
        out_shape=jax.ShapeDtypeStruct(d2r.shape, jnp.float32),
    )(d2r)


def _pack_bf16_pair(a, b):
    ha = jax.lax.bitcast_convert_type(
        a.astype(jnp.bfloat16).astype(jnp.float32), jnp.uint32)
    hb = jax.lax.bitcast_convert_type(
        b.astype(jnp.bfloat16).astype(jnp.float32), jnp.uint32)
    return ha | (hb >> 16)


def _tc_embed(x, pe, E1w, E1b, E2w, E2b, G1w, G1b, G2w, G2b,
              WT, WU, b1, bp1):
    N = x.shape[0]
    BN = 2000

    def body(x_ref, pe_ref, e1w, e1b, e2w, e2b, g1w, g1b, g2w, g2b,
             wt, wu, b1r, bp1r, h_ref, peh_ref, t_ref, u_ref):
        xin = jnp.concatenate([x_ref[...], pe_ref[...]], axis=1)
        t = _silu(jnp.dot(xin, e1w[...], preferred_element_type=jnp.float32)
                  + e1b[...])
        h = jnp.dot(t, e2w[...], preferred_element_type=jnp.float32) + e2b[...]
        h_ref[...] = h
        tp = _silu(jnp.dot(pe_ref[...], g1w[...], preferred_element_type=jnp.float32)
                   + g1b[...])
        peh = jnp.dot(tp, g2w[...], preferred_element_type=jnp.float32) + g2b[...]
        peh_ref[...] = peh
        z = jnp.concatenate([h, peh], axis=1)
        tm = jnp.dot(z, wt[:, :H], preferred_element_type=jnp.float32) + b1r[...]
        tpp = jnp.dot(z, wt[:, H:], preferred_element_type=jnp.float32) + bp1r[...]
        um = jnp.dot(z, wu[:, :H], preferred_element_type=jnp.float32)
        up = jnp.dot(z, wu[:, H:], preferred_element_type=jnp.float32)
        t_ref[...] = _pack_bf16_pair(tm, tpp)
        u_ref[...] = _pack_bf16_pair(um, up)

    full = lambda s: pl.BlockSpec(s, lambda i: (0, 0))
    return pl.pallas_call(
        body,
        grid=(N // BN,),
        in_specs=[
            pl.BlockSpec((BN, x.shape[1]), lambda i: (i, 0)),
            pl.BlockSpec((BN, pe.shape[1]), lambda i: (i, 0)),
            full(E1w.shape), full((1, H)), full(E2w.shape), full((1, H)),
            full(G1w.shape), full((1, H)), full(G2w.shape), full((1, H)),
            full(WT.shape), full(WU.shape), full((1, H)), full((1, H)),
        ],
        out_specs=[pl.BlockSpec((BN, H), lambda i: (i, 0)),
                   pl.BlockSpec((BN, H), lambda i: (i, 0)),
                   pl.BlockSpec((BN, H), lambda i: (i, 0)),
                   pl.BlockSpec((BN, H), lambda i: (i, 0))],
        out_shape=[jax.ShapeDtypeStruct((N, H), jnp.float32),
                   jax.ShapeDtypeStruct((N, H), jnp.float32),
                   jax.ShapeDtypeStruct((N, H), jnp.uint32),
                   jax.ShapeDtypeStruct((N, H), jnp.uint32)],
    )(x, pe, E1w, E1b[None, :], E2w, E2b[None, :],
      G1w, G1b[None, :], G2w, G2b[None, :],
      WT, WU, b1[None, :], bp1[None, :])


def _tc_edge(g1, g2, d2, vecs, W2, W2p):
    E = g1.shape[0]
    BE = 2000

    def body(g1_ref, g2_ref, d_ref, v_ref, w2, w2p, m_ref):
        dist = d_ref[...]
        g1w = g1_ref[...]
        g2w = g2_ref[...]
        hi = jnp.uint32(0xFFFF0000)
        unf = lambda u: jax.lax.bitcast_convert_type(u, jnp.float32)
        pre1 = unf(g1w & hi) + unf(g2w & hi) + dist * v_ref[0:1, :]
        pre1p = unf(g1w << 16) + unf(g2w << 16) + dist * v_ref[1:2, :]
        t = _silu(pre1)
        u = jnp.dot(t, w2[...], preferred_element_type=jnp.float32) + v_ref[2:3, :]
        m_ref[0] = _silu(u)
        tp = jnp.tanh(pre1p)
        up = jnp.dot(tp, w2p[...], preferred_element_type=jnp.float32) + v_ref[3:4, :]
        m_ref[1] = jnp.tanh(up)

    return pl.pallas_call(
        body,
        grid=(E // BE,),
        in_specs=[
            pl.BlockSpec((BE, H), lambda i: (i, 0)),
            pl.BlockSpec((BE, H), lambda i: (i, 0)),
            pl.BlockSpec((BE, 1), lambda i: (i, 0)),
            pl.BlockSpec(vecs.shape, lambda i: (0, 0)),
            pl.BlockSpec(W2.shape, lambda i: (0, 0)),
            pl.BlockSpec(W2p.shape, lambda i: (0, 0)),
        ],
        out_specs=pl.BlockSpec((2, BE, H), lambda i: (0, i, 0)),
        out_shape=jax.ShapeDtypeStruct((2, E, H), jnp.float32),
    )(g1, g2, d2, vecs, W2, W2p)


def _tc_update(h, pe_h, aggr, aggrp, V1, c1, V2, c2, P1, p1, P2, p2,
               nxt=None):
    N = h.shape[0]
    BN = 2000
    fused = nxt is not None

    def body(h_ref, pe_ref, a_ref, ap_ref, v1, c1r, v2, c2r, q1, p1r, q2, p2r,
             *rest):
        if fused:
            wt, wu, b1r, bp1r, hn_ref, pen_ref, t_ref, u_ref = rest
        else:
            hn_ref, pen_ref = rest
        cat = jnp.concatenate([h_ref[...], pe_ref[...], a_ref[...]], axis=1)
        z = _silu(jnp.dot(cat, v1[...], preferred_element_type=jnp.float32) + c1r[...])
        upd = jnp.dot(z, v2[...], preferred_element_type=jnp.float32) + c2r[...]
        hn = h_ref[...] + upd
        hn_ref[...] = hn
        catp = jnp.concatenate([pe_ref[...], ap_ref[...]], axis=1)
        zp = jnp.tanh(jnp.dot(catp, q1[...], preferred_element_type=jnp.float32) + p1r[...])
        updp = jnp.tanh(jnp.dot(zp, q2[...], preferred_element_type=jnp.float32) + p2r[...])
        pen = pe_ref[...] + updp
        pen_ref[...] = pen
        if fused:
            zz = jnp.concatenate([hn, pen], axis=1)
            tm = jnp.dot(zz, wt[:, :H], preferred_element_type=jnp.float32) + b1r[...]
            tpp = jnp.dot(zz, wt[:, H:], preferred_element_type=jnp.float32) + bp1r[...]
            um = jnp.dot(zz, wu[:, :H], preferred_element_type=jnp.float32)
            up = jnp.dot(zz, wu[:, H:], preferred_element_type=jnp.float32)
            t_ref[...] = _pack_bf16_pair(tm, tpp)
            u_ref[...] = _pack_bf16_pair(um, up)

    full = lambda s: pl.BlockSpec(s, lambda i: (0, 0))
    row = lambda: pl.BlockSpec((BN, H), lambda i: (i, 0))
    in_specs = [
        row(), row(), row(), row(),
        full(V1.shape), full((1, H)), full(V2.shape), full((1, H)),
        full(P1.shape), full((1, H)), full(P2.shape), full((1, H)),
    ]
    args = [h, pe_h, aggr, aggrp, V1, c1[None, :], V2, c2[None, :],
            P1, p1[None, :], P2, p2[None, :]]
    out_specs = [row(), row()]
    out_shape = [jax.ShapeDtypeStruct((N, H), jnp.float32),
                 jax.ShapeDtypeStruct((N, H), jnp.float32)]
    if fused:
        WT, WU, b1, bp1 = nxt
        in_specs += [full(WT.shape), full(WU.shape), full((1, H)), full((1, H))]
        args += [WT, WU, b1[None, :], bp1[None, :]]
        out_specs += [row(), row()]
        out_shape += [jax.ShapeDtypeStruct((N, H), jnp.uint32),
                      jax.ShapeDtypeStruct((N, H), jnp.uint32)]
    return pl.pallas_call(
        body,
        grid=(N // BN,),
        in_specs=in_specs,
        out_specs=out_specs,
        out_shape=out_shape,
    )(*args)


def _tc_final(h, batch2d, NB, Q1, q1, Q2, q2, R1, r1, R2p, r2p):
    N = h.shape[0]

    def body(h_ref, b_ref, w1, b1r, w2, b2r, w3, b3r, w4, b4r, out_ref):
        t = _silu(jnp.dot(h_ref[...], w1[...], preferred_element_type=jnp.float32)
                  + b1r[...])
        hpre = jnp.dot(t, w2[...], preferred_element_type=jnp.float32) + b2r[...]
        seg = lax.broadcasted_iota(jnp.int32, (NB, N), 0)
        oh = (b_ref[...] == seg).astype(jnp.float32)
        pooled = jnp.dot(oh, hpre, preferred_element_type=jnp.float32)
        tr = _silu(jnp.dot(pooled, w3[...], preferred_element_type=jnp.float32)
                   + b3r[...])
        out_ref[...] = jnp.dot(tr, w4[...], preferred_element_type=jnp.float32) + b4r[...]

    return pl.pallas_call(
        body,
        out_shape=jax.ShapeDtypeStruct((NB, H), jnp.float32),
    )(h, batch2d, Q1, q1[None, :], Q2, q2[None, :],
      R1, r1[None, :], R2p, r2p[None, :])


def kernel(x, pos, pe, params, edge_index, batch):
    N = x.shape[0]
    E = edge_index.shape[1]
    NB = 64

    send = edge_index[0].astype(jnp.int32)
    rec = edge_index[1].astype(jnp.int32)

    d2 = _sc_d2(pos[:, 0], pos[:, 1], pos[:, 2], send, rec)
    dist = _tc_dist(d2.reshape(E // H, H)).reshape(E, 1)

    layers = params['layers']
    L = len(layers)
    WTs, WUs, vecss, b1s, bp1s = [], [], [], [], []
    zblk = jnp.zeros((H, H), jnp.float32)
    for lp in layers:
        W1 = lp['message_mlp'][0]['w']
        b1 = lp['message_mlp'][0]['b']
        b2 = lp['message_mlp'][1]['b']
        Wp1 = lp['message_pos_mlp'][0]['w']
        bp1 = lp['message_pos_mlp'][0]['b']
        bp2 = lp['message_pos_mlp'][1]['b']
        WTs.append(jnp.concatenate([
            jnp.concatenate([W1[:H], zblk], axis=1),
            jnp.concatenate([W1[H:2 * H], Wp1[:H]], axis=1)], axis=0))
        WUs.append(jnp.concatenate([
            jnp.concatenate([W1[2 * H:3 * H], zblk], axis=1),
            jnp.concatenate([W1[3 * H:4 * H], Wp1[H:2 * H]], axis=1)], axis=0))
        zrow = jnp.zeros((H,), jnp.float32)
        vecss.append(jnp.stack([W1[4 * H], Wp1[2 * H], b2, bp2,
                                zrow, zrow, zrow, zrow], axis=0))
        b1s.append(b1)
        bp1s.append(bp1)

    emb = params['embed']
    embp = params['embed_pe']
    h, pe_h, T2, U2 = _tc_embed(
        x, pe,
        emb[0]['w'], emb[0]['b'], emb[1]['w'], emb[1]['b'],
        embp[0]['w'], embp[0]['b'], embp[1]['w'], embp[1]['b'],
        WTs[0], WUs[0], b1s[0], bp1s[0])

    NPAD = 10240
    zeros_nh = jnp.zeros((NPAD, H), jnp.float32)

    for li, lp in enumerate(layers):
        g1, g2 = _sc_gather(T2, U2, send, rec)
        M = _tc_edge(g1, g2, dist, vecss[li], lp['message_mlp'][1]['w'],
                     lp['message_pos_mlp'][1]['w'])
        A = _sc_scatter(M, rec, zeros_nh)

        nxt = None
        if li + 1 < L:
            nxt = (WTs[li + 1], WUs[li + 1], b1s[li + 1], bp1s[li + 1])
        res = _tc_update(h, pe_h, A[0, :N], A[1, :N],
                         lp['update_mlp'][0]['w'], lp['update_mlp'][0]['b'],
                         lp['update_mlp'][1]['w'], lp['update_mlp'][1]['b'],
                         lp['update_pos_mlp'][0]['w'], lp['update_pos_mlp'][0]['b'],
                         lp['update_pos_mlp'][1]['w'], lp['update_pos_mlp'][1]['b'],
                         nxt=nxt)
        if nxt is None:
            h, pe_h = res
        else:
            h, pe_h, T2, U2 = res

    pr = params['pre_readout']
    ro = params['readout']
    R2 = ro[1]['w']
    R2p = jnp.concatenate([R2, jnp.zeros((H, H - 1), jnp.float32)], axis=1)
    r2p = jnp.concatenate([ro[1]['b'], jnp.zeros((H - 1,), jnp.float32)], axis=0)
    batch2d = batch.astype(jnp.int32)[None, :]
    out = _tc_final(h, batch2d, NB,
                    pr[0]['w'], pr[0]['b'], pr[1]['w'], pr[1]['b'],
                    ro[0]['w'], ro[0]['b'], R2p, r2p)
    return out[:, 0]


# speedup vs baseline: 5.3916x; 1.0046x over previous
import functools

import jax
import jax.numpy as jnp
from jax import lax
from jax.experimental import pallas as pl
from jax.experimental.pallas import tpu as pltpu
from jax.experimental.pallas import tpu_sc as plsc

NC = 2
NS = 16
NW = NC * NS
LANE = 16

H = 128


def _mesh():
    return plsc.VectorSubcoreMesh(core_axis_name="c", subcore_axis_name="s",
                                  num_cores=NC, num_subcores=NS)


def _sc_d2(pxa, pya, pza, send, rec):
    N = pxa.shape[0]
    E = send.shape[0]
    EW = E // NW
    C = 80
    steps = EW // C

    @functools.partial(
        pl.kernel,
        out_type=jax.ShapeDtypeStruct((E,), jnp.float32),
        mesh=_mesh(),
        compiler_params=pltpu.CompilerParams(needs_layout_passes=False),
        scratch_types=[
            pltpu.VMEM((N,), jnp.float32),
            pltpu.VMEM((N,), jnp.float32),
            pltpu.VMEM((N,), jnp.float32),
            pltpu.VMEM((C,), jnp.int32),
            pltpu.VMEM((C,), jnp.int32),
            pltpu.VMEM((C,), jnp.float32),
        ],
    )
    def k(px_hbm, py_hbm, pz_hbm, send_hbm, rec_hbm, d2_hbm,
          px, py, pz, sidx, ridx, dbuf):
        wid = lax.axis_index("s") * NC + lax.axis_index("c")
        base = wid * EW
        pltpu.sync_copy(px_hbm, px)
        pltpu.sync_copy(py_hbm, py)
        pltpu.sync_copy(pz_hbm, pz)

        def chunk(i, carry):
            off = base + i * C
            pltpu.sync_copy(send_hbm.at[pl.ds(off, C)], sidx)
            pltpu.sync_copy(rec_hbm.at[pl.ds(off, C)], ridx)

            def sub(j, carry2):
                s16 = sidx[pl.ds(j * LANE, LANE)]
                r16 = ridx[pl.ds(j * LANE, LANE)]
                acc = jnp.zeros((LANE,), jnp.float32)
                for tab in (px, py, pz):
                    a = plsc.load_gather(tab, [s16])
                    b = plsc.load_gather(tab, [r16])
                    d = a - b
                    acc = acc + d * d
                dbuf[pl.ds(j * LANE, LANE)] = acc
                return carry2

            lax.fori_loop(0, C // LANE, sub, 0)
            pltpu.sync_copy(dbuf, d2_hbm.at[pl.ds(off, C)])
            return carry

        lax.fori_loop(0, steps, chunk, 0)

    return k(pxa, pya, pza, send, rec)


def _sc_gather(T2, U2, send, rec):
    N, D = T2.shape
    E = send.shape[0]
    EW = E // NW
    C = next(c for c in (80, 40, 16, 8)
             if EW % c == 0 and (EW // c) % 2 == 1)
    steps = EW // C
    assert C % 8 == 0 and EW % C == 0 and steps % 2 == 1 and steps >= 3
    pairs = steps // 2

    dt = T2.dtype

    @functools.partial(
        pl.kernel,
        out_type=(jax.ShapeDtypeStruct((E, D), dt),
                  jax.ShapeDtypeStruct((E, D), dt)),
        mesh=_mesh(),
        compiler_params=pltpu.CompilerParams(needs_layout_passes=False),
        scratch_types=[
            pltpu.VMEM((C,), jnp.int32),
            pltpu.VMEM((C,), jnp.int32),
            pltpu.VMEM((C,), jnp.int32),
            pltpu.VMEM((C,), jnp.int32),
            pltpu.VMEM((C, D), dt),
            pltpu.VMEM((C, D), dt),
            pltpu.VMEM((C, D), dt),
            pltpu.VMEM((C, D), dt),
            pltpu.SemaphoreType.DMA,
            pltpu.SemaphoreType.DMA,
            pltpu.SemaphoreType.DMA,
            pltpu.SemaphoreType.DMA,
            pltpu.SemaphoreType.DMA,
            pltpu.SemaphoreType.DMA,
        ],
    )
    def k(t_hbm, u_hbm, send_hbm, rec_hbm, g1_hbm, g2_hbm,
          sidx0, sidx1, ridx0, ridx1, r10, r11, r20, r21,
          isem0, isem1, gsem0, gsem1, wsem0, wsem1):
        wid = lax.axis_index("s") * NC + lax.axis_index("c")
        base = wid * EW
        sidx = (sidx0, sidx1)
        ridx = (ridx0, ridx1)
        r1 = (r10, r11)
        r2 = (r20, r21)
        isem = (isem0, isem1)
        gsem = (gsem0, gsem1)
        wsem = (wsem0, wsem1)

        def idx_issue(i, b):
            off = base + i * C
            pltpu.async_copy(send_hbm.at[pl.ds(off, C)], sidx[b], isem[b])
            pltpu.async_copy(rec_hbm.at[pl.ds(off, C)], ridx[b], isem[b])

        def idx_wait(b):
            pltpu.make_async_copy(send_hbm.at[pl.ds(0, C)], sidx[b], isem[b]).wait()
            pltpu.make_async_copy(rec_hbm.at[pl.ds(0, C)], ridx[b], isem[b]).wait()

        def gat_issue(b):
            pltpu.async_copy(t_hbm.at[sidx[b]], r1[b], gsem[b])
            pltpu.async_copy(u_hbm.at[ridx[b]], r2[b], gsem[b])

        def gat_wait(b):
            pltpu.make_async_copy(t_hbm.at[pl.ds(0, C)], r1[b], gsem[b]).wait()
            pltpu.make_async_copy(u_hbm.at[pl.ds(0, C)], r2[b], gsem[b]).wait()

        def wr_issue(i, b):
            off = base + i * C
            pltpu.async_copy(r1[b], g1_hbm.at[pl.ds(off, C)], wsem[b])
            pltpu.async_copy(r2[b], g2_hbm.at[pl.ds(off, C)], wsem[b])

        def wr_wait(b):
            pltpu.make_async_copy(r1[b], g1_hbm.at[pl.ds(0, C)], wsem[b]).wait()
            pltpu.make_async_copy(r2[b], g2_hbm.at[pl.ds(0, C)], wsem[b]).wait()

        idx_issue(0, 0)
        idx_issue(1, 1)
        idx_wait(0)
        gat_issue(0)

        def pair(kk, carry):
            i0 = 2 * kk

            @pl.when(kk > 0)
            def _():
                wr_wait(1)
            idx_wait(1)
            gat_issue(1)
            gat_wait(0)
            wr_issue(i0, 0)
            idx_issue(i0 + 2, 0)
            wr_wait(0)
            idx_wait(0)
            gat_issue(0)
            gat_wait(1)
            wr_issue(i0 + 1, 1)

            @pl.when(kk < pairs - 1)
            def _():
                idx_issue(i0 + 3, 1)
            return carry

        lax.fori_loop(0, pairs, pair, 0)
        gat_wait(0)
        wr_wait(1)
        wr_issue(steps - 1, 0)
        wr_wait(0)

    return k(T2, U2, send, rec)


def _sc_scatter(M, rec, zeros):
    _, E, D = M.shape
    NP = zeros.shape[0]
    ET = E // NS
    C = next(c for c in (80, 40, 16, 8)
             if ET % c == 0 and (ET // c) % 4 == 2)
    steps = ET // C
    NT = NP // NS
    assert steps % 4 == 2 and steps >= 6
    quads = (steps - 2) // 4

    @functools.partial(
        pl.kernel,
        out_type=jax.ShapeDtypeStruct((2, NP, D), jnp.float32),
        mesh=_mesh(),
        compiler_params=pltpu.CompilerParams(needs_layout_passes=False),
        scratch_types=[
            pltpu.VMEM((C,), jnp.int32),
            pltpu.VMEM((C,), jnp.int32),
            pltpu.VMEM((C,), jnp.int32),
            pltpu.VMEM((C,), jnp.int32),
            pltpu.VMEM((C, D), jnp.float32),
            pltpu.VMEM((C, D), jnp.float32),
            pltpu.VMEM((C, D), jnp.float32),
            pltpu.VMEM((C, D), jnp.float32),
            pltpu.VMEM_SHARED((NP, D), jnp.float32),
            pltpu.SemaphoreType.DMA,
            pltpu.SemaphoreType.DMA,
            pltpu.SemaphoreType.DMA,
            pltpu.SemaphoreType.DMA,
            pltpu.SemaphoreType.DMA,
            pltpu.SemaphoreType.DMA,
            pltpu.SemaphoreType.DMA,
            pltpu.SemaphoreType.DMA,
        ],
    )
    def k(m_hbm, rec_hbm, z_hbm, out_hbm,
          ridx0, ridx1, ridx2, ridx3, rows0, rows1, rows2, rows3,
          acc, lsem0, lsem1, lsem2, lsem3, ssem0, ssem1, ssem2, ssem3):
        cid = lax.axis_index("c")
        sid = lax.axis_index("s")
        nb = pl.multiple_of(sid * NT, 8)
        base = sid * ET
        ridx = (ridx0, ridx1, ridx2, ridx3)
        rows = (rows0, rows1, rows2, rows3)
        lsem = (lsem0, lsem1, lsem2, lsem3)
        ssem = (ssem0, ssem1, ssem2, ssem3)

        def ld_issue(i, b):
            off = base + i * C
            pltpu.async_copy(rec_hbm.at[pl.ds(off, C)], ridx[b], lsem[b])
            pltpu.async_copy(m_hbm.at[cid, pl.ds(off, C)], rows[b], lsem[b])

        def ld_wait(b):
            pltpu.make_async_copy(rec_hbm.at[pl.ds(0, C)], ridx[b], lsem[b]).wait()
            pltpu.make_async_copy(m_hbm.at[cid, pl.ds(0, C)], rows[b], lsem[b]).wait()

        def scat_issue(b):
            pltpu.async_copy(rows[b], acc.at[ridx[b]], ssem[b], add=True)

        def scat_wait(b):
            pltpu.make_async_copy(rows[b], acc.at[pl.ds(0, C)], ssem[b]).wait()

        ld_issue(0, 0)
        pltpu.sync_copy(z_hbm.at[pl.ds(nb, NT)], acc.at[pl.ds(nb, NT)])
        plsc.subcore_barrier()
        ld_issue(1, 1)
        ld_issue(2, 2)

        def step(j, b, kk, first_quad, last_quad):
            ld_wait(b)
            scat_issue(b)
            prev = (b - 1) % 4
            if first_quad is None:
                scat_wait(prev)
            else:
                @pl.when(kk > 0)
                def _():
                    scat_wait(prev)
            nxt = (b + 3) % 4
            if last_quad is None:
                ld_issue(j + 3, nxt)
            elif last_quad:
                @pl.when(kk < quads - 1)
                def _():
                    ld_issue(j + 3, nxt)

        def quad(kk, carry):
            j0 = 4 * kk
            step(j0 + 0, 0, kk, True, None)
            step(j0 + 1, 1, kk, None, None)
            step(j0 + 2, 2, kk, None, None)
            step(j0 + 3, 3, kk, None, True)
            return carry

        lax.fori_loop(0, quads, quad, 0)
        ld_wait(0)
        scat_issue(0)
        scat_wait(3)
        ld_wait(1)
        scat_issue(1)
        scat_wait(0)
        scat_wait(1)
        plsc.subcore_barrier()
        pltpu.sync_copy(acc.at[pl.ds(nb, NT)],
                        out_hbm.at[cid, pl.ds(nb, NT)])

    return k(M, rec, zeros)


_silu = jax.nn.silu


def _tc_dist(d2r):
    def body(d_ref, o_ref):
        o_ref[...] = jnp.sqrt(d_ref[...])

    return pl.pallas_call(
        body,
        out_shape=jax.ShapeDtypeStruct(d2r.shape, jnp.float32),
    )(d2r)


def _pack_bf16_pair(a, b):
    ha = jax.lax.bitcast_convert_type(
        a.astype(jnp.bfloat16).astype(jnp.float32), jnp.uint32)
    hb = jax.lax.bitcast_convert_type(
        b.astype(jnp.bfloat16).astype(jnp.float32), jnp.uint32)
    return ha | (hb >> 16)


def _tc_embed(x, pe, E1w, E1b, E2w, E2b, G1w, G1b, G2w, G2b,
              WT, WU, b1, bp1):
    N = x.shape[0]
    BN = 2000

    def body(x_ref, pe_ref, e1w, e1b, e2w, e2b, g1w, g1b, g2w, g2b,
             wt, wu, b1r, bp1r, h_ref, peh_ref, t_ref, u_ref):
        xin = jnp.concatenate([x_ref[...], pe_ref[...]], axis=1)
        t = _silu(jnp.dot(xin, e1w[...], preferred_element_type=jnp.float32)
                  + e1b[...])
        h = jnp.dot(t, e2w[...], preferred_element_type=jnp.float32) + e2b[...]
        h_ref[...] = h
        tp = _silu(jnp.dot(pe_ref[...], g1w[...], preferred_element_type=jnp.float32)
                   + g1b[...])
        peh = jnp.dot(tp, g2w[...], preferred_element_type=jnp.float32) + g2b[...]
        peh_ref[...] = peh
        z = jnp.concatenate([h, peh], axis=1)
        tm = jnp.dot(z, wt[:, :H], preferred_element_type=jnp.float32) + b1r[...]
        tpp = jnp.dot(z, wt[:, H:], preferred_element_type=jnp.float32) + bp1r[...]
        um = jnp.dot(z, wu[:, :H], preferred_element_type=jnp.float32)
        up = jnp.dot(z, wu[:, H:], preferred_element_type=jnp.float32)
        t_ref[...] = _pack_bf16_pair(tm, tpp)
        u_ref[...] = _pack_bf16_pair(um, up)

    full = lambda s: pl.BlockSpec(s, lambda i: (0, 0))
    return pl.pallas_call(
        body,
        grid=(N // BN,),
        in_specs=[
            pl.BlockSpec((BN, x.shape[1]), lambda i: (i, 0)),
            pl.BlockSpec((BN, pe.shape[1]), lambda i: (i, 0)),
            full(E1w.shape), full((1, H)), full(E2w.shape), full((1, H)),
            full(G1w.shape), full((1, H)), full(G2w.shape), full((1, H)),
            full(WT.shape), full(WU.shape), full((1, H)), full((1, H)),
        ],
        out_specs=[pl.BlockSpec((BN, H), lambda i: (i, 0)),
                   pl.BlockSpec((BN, H), lambda i: (i, 0)),
                   pl.BlockSpec((BN, H), lambda i: (i, 0)),
                   pl.BlockSpec((BN, H), lambda i: (i, 0))],
        out_shape=[jax.ShapeDtypeStruct((N, H), jnp.float32),
                   jax.ShapeDtypeStruct((N, H), jnp.float32),
                   jax.ShapeDtypeStruct((N, H), jnp.uint32),
                   jax.ShapeDtypeStruct((N, H), jnp.uint32)],
    )(x, pe, E1w, E1b[None, :], E2w, E2b[None, :],
      G1w, G1b[None, :], G2w, G2b[None, :],
      WT, WU, b1[None, :], bp1[None, :])


def _tc_edge(g1, g2, d2, vecs, W2, W2p):
    E = g1.shape[0]
    BE = 2000

    def body(g1_ref, g2_ref, d_ref, v_ref, w2, w2p, m_ref):
        dist = d_ref[...]
        g1w = g1_ref[...]
        g2w = g2_ref[...]
        hi = jnp.uint32(0xFFFF0000)
        unf = lambda u: jax.lax.bitcast_convert_type(u, jnp.float32)
        pre1 = unf(g1w & hi) + unf(g2w & hi) + dist * v_ref[0:1, :]
        pre1p = unf(g1w << 16) + unf(g2w << 16) + dist * v_ref[1:2, :]
        t = _silu(pre1)
        u = jnp.dot(t, w2[...], preferred_element_type=jnp.float32) + v_ref[2:3, :]
        m_ref[0] = _silu(u)
        tp = jnp.tanh(pre1p)
        up = jnp.dot(tp, w2p[...], preferred_element_type=jnp.float32) + v_ref[3:4, :]
        m_ref[1] = jnp.tanh(up)

    return pl.pallas_call(
        body,
        grid=(E // BE,),
        in_specs=[
            pl.BlockSpec((BE, H), lambda i: (i, 0)),
            pl.BlockSpec((BE, H), lambda i: (i, 0)),
            pl.BlockSpec((BE, 1), lambda i: (i, 0)),
            pl.BlockSpec(vecs.shape, lambda i: (0, 0)),
            pl.BlockSpec(W2.shape, lambda i: (0, 0)),
            pl.BlockSpec(W2p.shape, lambda i: (0, 0)),
        ],
        out_specs=pl.BlockSpec((2, BE, H), lambda i: (0, i, 0)),
        out_shape=jax.ShapeDtypeStruct((2, E, H), jnp.float32),
    )(g1, g2, d2, vecs, W2, W2p)


def _tc_update(h, pe_h, Aa, Ab, V1, c1, V2, c2, P1, p1, P2, p2,
               nxt=None):
    N = h.shape[0]
    BN = 2000
    fused = nxt is not None

    def body(h_ref, pe_ref, aa_ref, ab_ref, v1, c1r, v2, c2r, q1, p1r, q2, p2r,
             *rest):
        if fused:
            wt, wu, b1r, bp1r, hn_ref, pen_ref, t_ref, u_ref = rest
        else:
            hn_ref, pen_ref = rest
        a = aa_ref[0] + ab_ref[0]
        ap = aa_ref[1] + ab_ref[1]
        cat = jnp.concatenate([h_ref[...], pe_ref[...], a], axis=1)
        z = _silu(jnp.dot(cat, v1[...], preferred_element_type=jnp.float32) + c1r[...])
        upd = jnp.dot(z, v2[...], preferred_element_type=jnp.float32) + c2r[...]
        hn = h_ref[...] + upd
        hn_ref[...] = hn
        catp = jnp.concatenate([pe_ref[...], ap], axis=1)
        zp = jnp.tanh(jnp.dot(catp, q1[...], preferred_element_type=jnp.float32) + p1r[...])
        updp = jnp.tanh(jnp.dot(zp, q2[...], preferred_element_type=jnp.float32) + p2r[...])
        pen = pe_ref[...] + updp
        pen_ref[...] = pen
        if fused:
            zz = jnp.concatenate([hn, pen], axis=1)
            tm = jnp.dot(zz, wt[:, :H], preferred_element_type=jnp.float32) + b1r[...]
            tpp = jnp.dot(zz, wt[:, H:], preferred_element_type=jnp.float32) + bp1r[...]
            um = jnp.dot(zz, wu[:, :H], preferred_element_type=jnp.float32)
            up = jnp.dot(zz, wu[:, H:], preferred_element_type=jnp.float32)
            t_ref[...] = _pack_bf16_pair(tm, tpp)
            u_ref[...] = _pack_bf16_pair(um, up)

    full = lambda s: pl.BlockSpec(s, lambda i: (0, 0))
    row = lambda: pl.BlockSpec((BN, H), lambda i: (i, 0))
    arow = lambda: pl.BlockSpec((2, BN, H), lambda i: (0, i, 0))
    in_specs = [
        row(), row(), arow(), arow(),
        full(V1.shape), full((1, H)), full(V2.shape), full((1, H)),
        full(P1.shape), full((1, H)), full(P2.shape), full((1, H)),
    ]
    args = [h, pe_h, Aa, Ab, V1, c1[None, :], V2, c2[None, :],
            P1, p1[None, :], P2, p2[None, :]]
    out_specs = [row(), row()]
    out_shape = [jax.ShapeDtypeStruct((N, H), jnp.float32),
                 jax.ShapeDtypeStruct((N, H), jnp.float32)]
    if fused:
        WT, WU, b1, bp1 = nxt
        in_specs += [full(WT.shape), full(WU.shape), full((1, H)), full((1, H))]
        args += [WT, WU, b1[None, :], bp1[None, :]]
        out_specs += [row(), row()]
        out_shape += [jax.ShapeDtypeStruct((N, H), jnp.uint32),
                      jax.ShapeDtypeStruct((N, H), jnp.uint32)]
    return pl.pallas_call(
        body,
        grid=(N // BN,),
        in_specs=in_specs,
        out_specs=out_specs,
        out_shape=out_shape,
    )(*args)


def _tc_final(h, batch2d, NB, Q1, q1, Q2, q2, R1, r1, R2p, r2p):
    N = h.shape[0]

    def body(h_ref, b_ref, w1, b1r, w2, b2r, w3, b3r, w4, b4r, out_ref):
        t = _silu(jnp.dot(h_ref[...], w1[...], preferred_element_type=jnp.float32)
                  + b1r[...])
        hpre = jnp.dot(t, w2[...], preferred_element_type=jnp.float32) + b2r[...]
        seg = lax.broadcasted_iota(jnp.int32, (NB, N), 0)
        oh = (b_ref[...] == seg).astype(jnp.float32)
        pooled = jnp.dot(oh, hpre, preferred_element_type=jnp.float32)
        tr = _silu(jnp.dot(pooled, w3[...], preferred_element_type=jnp.float32)
                   + b3r[...])
        out_ref[...] = jnp.dot(tr, w4[...], preferred_element_type=jnp.float32) + b4r[...]

    return pl.pallas_call(
        body,
        out_shape=jax.ShapeDtypeStruct((NB, H), jnp.float32),
    )(h, batch2d, Q1, q1[None, :], Q2, q2[None, :],
      R1, r1[None, :], R2p, r2p[None, :])


def kernel(x, pos, pe, params, edge_index, batch):
    N = x.shape[0]
    E = edge_index.shape[1]
    NB = 64

    send = edge_index[0].astype(jnp.int32)
    rec = edge_index[1].astype(jnp.int32)

    d2 = _sc_d2(pos[:, 0], pos[:, 1], pos[:, 2], send, rec)
    dist = _tc_dist(d2.reshape(E // H, H)).reshape(E, 1)

    layers = params['layers']
    L = len(layers)
    WTs, WUs, vecss, b1s, bp1s = [], [], [], [], []
    zblk = jnp.zeros((H, H), jnp.float32)
    for lp in layers:
        W1 = lp['message_mlp'][0]['w']
        b1 = lp['message_mlp'][0]['b']
        b2 = lp['message_mlp'][1]['b']
        Wp1 = lp['message_pos_mlp'][0]['w']
        bp1 = lp['message_pos_mlp'][0]['b']
        bp2 = lp['message_pos_mlp'][1]['b']
        WTs.append(jnp.concatenate([
            jnp.concatenate([W1[:H], zblk], axis=1),
            jnp.concatenate([W1[H:2 * H], Wp1[:H]], axis=1)], axis=0))
        WUs.append(jnp.concatenate([
            jnp.concatenate([W1[2 * H:3 * H], zblk], axis=1),
            jnp.concatenate([W1[3 * H:4 * H], Wp1[H:2 * H]], axis=1)], axis=0))
        zrow = jnp.zeros((H,), jnp.float32)
        vecss.append(jnp.stack([W1[4 * H], Wp1[2 * H], b2, bp2,
                                zrow, zrow, zrow, zrow], axis=0))
        b1s.append(b1)
        bp1s.append(bp1)

    emb = params['embed']
    embp = params['embed_pe']
    h, pe_h, T2, U2 = _tc_embed(
        x, pe,
        emb[0]['w'], emb[0]['b'], emb[1]['w'], emb[1]['b'],
        embp[0]['w'], embp[0]['b'], embp[1]['w'], embp[1]['b'],
        WTs[0], WUs[0], b1s[0], bp1s[0])

    NPAD = 10240
    zeros_nh = jnp.zeros((NPAD, H), jnp.float32)

    E2 = E // 2
    sendA, sendB = send[:E2], send[E2:]
    recA, recB = rec[:E2], rec[E2:]
    distA, distB = dist[:E2], dist[E2:]

    for li, lp in enumerate(layers):
        W2 = lp['message_mlp'][1]['w']
        Wp2 = lp['message_pos_mlp'][1]['w']
        g1A, g2A = _sc_gather(T2, U2, sendA, recA)
        MA = _tc_edge(g1A, g2A, distA, vecss[li], W2, Wp2)
        g1B, g2B = _sc_gather(T2, U2, sendB, recB)
        Aa = _sc_scatter(MA, recA, zeros_nh)
        MB = _tc_edge(g1B, g2B, distB, vecss[li], W2, Wp2)
        Ab = _sc_scatter(MB, recB, zeros_nh)

        nxt = None
        if li + 1 < L:
            nxt = (WTs[li + 1], WUs[li + 1], b1s[li + 1], bp1s[li + 1])
        res = _tc_update(h, pe_h, Aa[:, :N], Ab[:, :N],
                         lp['update_mlp'][0]['w'], lp['update_mlp'][0]['b'],
                         lp['update_mlp'][1]['w'], lp['update_mlp'][1]['b'],
                         lp['update_pos_mlp'][0]['w'], lp['update_pos_mlp'][0]['b'],
                         lp['update_pos_mlp'][1]['w'], lp['update_pos_mlp'][1]['b'],
                         nxt=nxt)
        if nxt is None:
            h, pe_h = res
        else:
            h, pe_h, T2, U2 = res

    pr = params['pre_readout']
    ro = params['readout']
    R2 = ro[1]['w']
    R2p = jnp.concatenate([R2, jnp.zeros((H, H - 1), jnp.float32)], axis=1)
    r2p = jnp.concatenate([ro[1]['b'], jnp.zeros((H - 1,), jnp.float32)], axis=0)
    batch2d = batch.astype(jnp.int32)[None, :]
    out = _tc_final(h, batch2d, NB,
                    pr[0]['w'], pr[0]['b'], pr[1]['w'], pr[1]['b'],
                    ro[0]['w'], ro[0]['b'], R2p, r2p)
    return out[:, 0]


# speedup vs baseline: 6.1723x; 1.1448x over previous
import functools

import jax
import jax.numpy as jnp
from jax import lax
from jax.experimental import pallas as pl
from jax.experimental.pallas import tpu as pltpu
from jax.experimental.pallas import tpu_sc as plsc

NC = 2
NS = 16
NW = NC * NS
LANE = 16

H = 128


def _mesh():
    return plsc.VectorSubcoreMesh(core_axis_name="c", subcore_axis_name="s",
                                  num_cores=NC, num_subcores=NS)


def _sc_d2(pxa, pya, pza, send, rec):
    N = pxa.shape[0]
    E = send.shape[0]
    EW = E // NW
    C = 80
    steps = EW // C

    @functools.partial(
        pl.kernel,
        out_type=jax.ShapeDtypeStruct((E,), jnp.float32),
        mesh=_mesh(),
        compiler_params=pltpu.CompilerParams(needs_layout_passes=False),
        scratch_types=[
            pltpu.VMEM((N,), jnp.float32),
            pltpu.VMEM((N,), jnp.float32),
            pltpu.VMEM((N,), jnp.float32),
            pltpu.VMEM((C,), jnp.int32),
            pltpu.VMEM((C,), jnp.int32),
            pltpu.VMEM((C,), jnp.float32),
        ],
    )
    def k(px_hbm, py_hbm, pz_hbm, send_hbm, rec_hbm, d2_hbm,
          px, py, pz, sidx, ridx, dbuf):
        wid = lax.axis_index("s") * NC + lax.axis_index("c")
        base = wid * EW
        pltpu.sync_copy(px_hbm, px)
        pltpu.sync_copy(py_hbm, py)
        pltpu.sync_copy(pz_hbm, pz)

        def chunk(i, carry):
            off = base + i * C
            pltpu.sync_copy(send_hbm.at[pl.ds(off, C)], sidx)
            pltpu.sync_copy(rec_hbm.at[pl.ds(off, C)], ridx)

            def sub(j, carry2):
                s16 = sidx[pl.ds(j * LANE, LANE)]
                r16 = ridx[pl.ds(j * LANE, LANE)]
                acc = jnp.zeros((LANE,), jnp.float32)
                for tab in (px, py, pz):
                    a = plsc.load_gather(tab, [s16])
                    b = plsc.load_gather(tab, [r16])
                    d = a - b
                    acc = acc + d * d
                dbuf[pl.ds(j * LANE, LANE)] = acc
                return carry2

            lax.fori_loop(0, C // LANE, sub, 0)
            pltpu.sync_copy(dbuf, d2_hbm.at[pl.ds(off, C)])
            return carry

        lax.fori_loop(0, steps, chunk, 0)

    return k(pxa, pya, pza, send, rec)


def _sc_gather(TU, eidx):
    _, NP2, D = TU.shape
    E = eidx.shape[0] // 2
    EW = E // NS
    C = next(c for c in (80, 40, 16, 8)
             if EW % c == 0 and (EW // c) % 2 == 1)
    steps = EW // C
    assert C % 8 == 0 and steps % 2 == 1 and steps >= 3
    pairs = steps // 2
    NT2 = NP2 // NS
    dt = TU.dtype

    @functools.partial(
        pl.kernel,
        out_type=jax.ShapeDtypeStruct((2, E, D), dt),
        mesh=_mesh(),
        compiler_params=pltpu.CompilerParams(needs_layout_passes=False),
        scratch_types=[
            pltpu.VMEM((C,), jnp.int32),
            pltpu.VMEM((C,), jnp.int32),
            pltpu.VMEM((C, D), dt),
            pltpu.VMEM((C, D), dt),
            pltpu.VMEM_SHARED((NP2, D), dt),
            pltpu.SemaphoreType.DMA,
            pltpu.SemaphoreType.DMA,
            pltpu.SemaphoreType.DMA,
            pltpu.SemaphoreType.DMA,
            pltpu.SemaphoreType.DMA,
            pltpu.SemaphoreType.DMA,
        ],
    )
    def k(tu_hbm, eidx_hbm, g_hbm,
          idx0, idx1, r0, r1, table,
          isem0, isem1, gsem0, gsem1, wsem0, wsem1):
        cid = lax.axis_index("c")
        sid = lax.axis_index("s")
        nb = pl.multiple_of(sid * NT2, 8)
        base = sid * EW
        idx = (idx0, idx1)
        rr = (r0, r1)
        isem = (isem0, isem1)
        gsem = (gsem0, gsem1)
        wsem = (wsem0, wsem1)

        def idx_issue(i, b):
            off = cid * E + base + i * C
            pltpu.async_copy(eidx_hbm.at[pl.ds(off, C)], idx[b], isem[b])

        def idx_wait(b):
            pltpu.make_async_copy(eidx_hbm.at[pl.ds(0, C)], idx[b],
                                  isem[b]).wait()

        def gat_issue(b):
            pltpu.async_copy(table.at[idx[b]], rr[b], gsem[b])

        def gat_wait(b):
            pltpu.make_async_copy(table.at[pl.ds(0, C)], rr[b], gsem[b]).wait()

        def wr_issue(i, b):
            off = base + i * C
            pltpu.async_copy(rr[b], g_hbm.at[cid, pl.ds(off, C)], wsem[b])

        def wr_wait(b):
            pltpu.make_async_copy(rr[b], g_hbm.at[cid, pl.ds(0, C)],
                                  wsem[b]).wait()

        pltpu.sync_copy(tu_hbm.at[cid, pl.ds(nb, NT2)],
                        table.at[pl.ds(nb, NT2)])
        idx_issue(0, 0)
        idx_issue(1, 1)
        plsc.subcore_barrier()
        idx_wait(0)
        gat_issue(0)

        def pair(kk, carry):
            i0 = 2 * kk

            @pl.when(kk > 0)
            def _():
                wr_wait(1)
            idx_wait(1)
            gat_issue(1)
            gat_wait(0)
            wr_issue(i0, 0)
            idx_issue(i0 + 2, 0)
            wr_wait(0)
            idx_wait(0)
            gat_issue(0)
            gat_wait(1)
            wr_issue(i0 + 1, 1)

            @pl.when(kk < pairs - 1)
            def _():
                idx_issue(i0 + 3, 1)
            return carry

        lax.fori_loop(0, pairs, pair, 0)
        gat_wait(0)
        wr_wait(1)
        wr_issue(steps - 1, 0)
        wr_wait(0)

    return k(TU, eidx)


def _sc_scatter(M, rec, zeros):
    _, E, D = M.shape
    NP = zeros.shape[0]
    ET = E // NS
    C = next(c for c in (80, 40, 16, 8)
             if ET % c == 0 and (ET // c) % 4 == 2)
    steps = ET // C
    NT = NP // NS
    assert steps % 4 == 2 and steps >= 6
    quads = (steps - 2) // 4

    @functools.partial(
        pl.kernel,
        out_type=jax.ShapeDtypeStruct((2, NP, D), jnp.float32),
        mesh=_mesh(),
        compiler_params=pltpu.CompilerParams(needs_layout_passes=False),
        scratch_types=[
            pltpu.VMEM((C,), jnp.int32),
            pltpu.VMEM((C,), jnp.int32),
            pltpu.VMEM((C,), jnp.int32),
            pltpu.VMEM((C,), jnp.int32),
            pltpu.VMEM((C, D), jnp.float32),
            pltpu.VMEM((C, D), jnp.float32),
            pltpu.VMEM((C, D), jnp.float32),
            pltpu.VMEM((C, D), jnp.float32),
            pltpu.VMEM_SHARED((NP, D), jnp.float32),
            pltpu.SemaphoreType.DMA,
            pltpu.SemaphoreType.DMA,
            pltpu.SemaphoreType.DMA,
            pltpu.SemaphoreType.DMA,
            pltpu.SemaphoreType.DMA,
            pltpu.SemaphoreType.DMA,
            pltpu.SemaphoreType.DMA,
            pltpu.SemaphoreType.DMA,
        ],
    )
    def k(m_hbm, rec_hbm, z_hbm, out_hbm,
          ridx0, ridx1, ridx2, ridx3, rows0, rows1, rows2, rows3,
          acc, lsem0, lsem1, lsem2, lsem3, ssem0, ssem1, ssem2, ssem3):
        cid = lax.axis_index("c")
        sid = lax.axis_index("s")
        nb = pl.multiple_of(sid * NT, 8)
        base = sid * ET
        ridx = (ridx0, ridx1, ridx2, ridx3)
        rows = (rows0, rows1, rows2, rows3)
        lsem = (lsem0, lsem1, lsem2, lsem3)
        ssem = (ssem0, ssem1, ssem2, ssem3)

        def ld_issue(i, b):
            off = base + i * C
            pltpu.async_copy(rec_hbm.at[pl.ds(off, C)], ridx[b], lsem[b])
            pltpu.async_copy(m_hbm.at[cid, pl.ds(off, C)], rows[b], lsem[b])

        def ld_wait(b):
            pltpu.make_async_copy(rec_hbm.at[pl.ds(0, C)], ridx[b], lsem[b]).wait()
            pltpu.make_async_copy(m_hbm.at[cid, pl.ds(0, C)], rows[b], lsem[b]).wait()

        def scat_issue(b):
            pltpu.async_copy(rows[b], acc.at[ridx[b]], ssem[b], add=True)

        def scat_wait(b):
            pltpu.make_async_copy(rows[b], acc.at[pl.ds(0, C)], ssem[b]).wait()

        ld_issue(0, 0)
        pltpu.sync_copy(z_hbm.at[pl.ds(nb, NT)], acc.at[pl.ds(nb, NT)])
        plsc.subcore_barrier()
        ld_issue(1, 1)
        ld_issue(2, 2)

        def step(j, b, kk, first_quad, last_quad):
            ld_wait(b)
            scat_issue(b)
            prev = (b - 1) % 4
            if first_quad is None:
                scat_wait(prev)
            else:
                @pl.when(kk > 0)
                def _():
                    scat_wait(prev)
            nxt = (b + 3) % 4
            if last_quad is None:
                ld_issue(j + 3, nxt)
            elif last_quad:
                @pl.when(kk < quads - 1)
                def _():
                    ld_issue(j + 3, nxt)

        def quad(kk, carry):
            j0 = 4 * kk
            step(j0 + 0, 0, kk, True, None)
            step(j0 + 1, 1, kk, None, None)
            step(j0 + 2, 2, kk, None, None)
            step(j0 + 3, 3, kk, None, True)
            return carry

        lax.fori_loop(0, quads, quad, 0)
        ld_wait(0)
        scat_issue(0)
        scat_wait(3)
        ld_wait(1)
        scat_issue(1)
        scat_wait(0)
        scat_wait(1)
        plsc.subcore_barrier()
        pltpu.sync_copy(acc.at[pl.ds(nb, NT)],
                        out_hbm.at[cid, pl.ds(nb, NT)])

    return k(M, rec, zeros)


_silu = jax.nn.silu


def _tc_dist(d2r):
    def body(d_ref, o_ref):
        o_ref[...] = jnp.sqrt(d_ref[...])

    return pl.pallas_call(
        body,
        out_shape=jax.ShapeDtypeStruct(d2r.shape, jnp.float32),
    )(d2r)


def _pack_bf16_pair(a, b):
    ha = jax.lax.bitcast_convert_type(
        a.astype(jnp.bfloat16).astype(jnp.float32), jnp.uint32)
    hb = jax.lax.bitcast_convert_type(
        b.astype(jnp.bfloat16).astype(jnp.float32), jnp.uint32)
    return ha | (hb >> 16)


def _tc_embed(x, pe, E1w, E1b, E2w, E2b, G1w, G1b, G2w, G2b,
              WT, WU, b1, bp1):
    N = x.shape[0]
    BN = 2000

    def body(x_ref, pe_ref, e1w, e1b, e2w, e2b, g1w, g1b, g2w, g2b,
             wt, wu, b1r, bp1r, h_ref, peh_ref, t_ref, u_ref):
        xin = jnp.concatenate([x_ref[...], pe_ref[...]], axis=1)
        t = _silu(jnp.dot(xin, e1w[...], preferred_element_type=jnp.float32)
                  + e1b[...])
        h = jnp.dot(t, e2w[...], preferred_element_type=jnp.float32) + e2b[...]
        h_ref[...] = h
        tp = _silu(jnp.dot(pe_ref[...], g1w[...], preferred_element_type=jnp.float32)
                   + g1b[...])
        peh = jnp.dot(tp, g2w[...], preferred_element_type=jnp.float32) + g2b[...]
        peh_ref[...] = peh
        z = jnp.concatenate([h, peh], axis=1)
        tm = jnp.dot(z, wt[:, :H], preferred_element_type=jnp.float32) + b1r[...]
        tpp = jnp.dot(z, wt[:, H:], preferred_element_type=jnp.float32) + bp1r[...]
        um = jnp.dot(z, wu[:, :H], preferred_element_type=jnp.float32)
        up = jnp.dot(z, wu[:, H:], preferred_element_type=jnp.float32)
        t_ref[...] = _pack_bf16_pair(tm, tpp)
        u_ref[...] = _pack_bf16_pair(um, up)

    full = lambda s: pl.BlockSpec(s, lambda i: (0, 0))
    return pl.pallas_call(
        body,
        grid=(N // BN,),
        in_specs=[
            pl.BlockSpec((BN, x.shape[1]), lambda i: (i, 0)),
            pl.BlockSpec((BN, pe.shape[1]), lambda i: (i, 0)),
            full(E1w.shape), full((1, H)), full(E2w.shape), full((1, H)),
            full(G1w.shape), full((1, H)), full(G2w.shape), full((1, H)),
            full(WT.shape), full(WU.shape), full((1, H)), full((1, H)),
        ],
        out_specs=[pl.BlockSpec((BN, H), lambda i: (i, 0)),
                   pl.BlockSpec((BN, H), lambda i: (i, 0)),
                   pl.BlockSpec((BN, H), lambda i: (i, 0)),
                   pl.BlockSpec((BN, H), lambda i: (i, 0))],
        out_shape=[jax.ShapeDtypeStruct((N, H), jnp.float32),
                   jax.ShapeDtypeStruct((N, H), jnp.float32),
                   jax.ShapeDtypeStruct((N, H), jnp.uint32),
                   jax.ShapeDtypeStruct((N, H), jnp.uint32)],
    )(x, pe, E1w, E1b[None, :], E2w, E2b[None, :],
      G1w, G1b[None, :], G2w, G2b[None, :],
      WT, WU, b1[None, :], bp1[None, :])


def _tc_edge(G, d2, vecs, W2, W2p):
    E = G.shape[1]
    BE = 2000

    def body(g_ref, d_ref, v_ref, w2, w2p, m_ref):
        dist = d_ref[...]
        g1w = g_ref[0]
        g2w = g_ref[1]
        hi = jnp.uint32(0xFFFF0000)
        unf = lambda u: jax.lax.bitcast_convert_type(u, jnp.float32)
        pre1 = unf(g1w & hi) + unf(g2w & hi) + dist * v_ref[0:1, :]
        pre1p = unf(g1w << 16) + unf(g2w << 16) + dist * v_ref[1:2, :]
        t = _silu(pre1)
        u = jnp.dot(t, w2[...], preferred_element_type=jnp.float32) + v_ref[2:3, :]
        m_ref[0] = _silu(u)
        tp = jnp.tanh(pre1p)
        up = jnp.dot(tp, w2p[...], preferred_element_type=jnp.float32) + v_ref[3:4, :]
        m_ref[1] = jnp.tanh(up)

    return pl.pallas_call(
        body,
        grid=(E // BE,),
        in_specs=[
            pl.BlockSpec((2, BE, H), lambda i: (0, i, 0)),
            pl.BlockSpec((BE, 1), lambda i: (i, 0)),
            pl.BlockSpec(vecs.shape, lambda i: (0, 0)),
            pl.BlockSpec(W2.shape, lambda i: (0, 0)),
            pl.BlockSpec(W2p.shape, lambda i: (0, 0)),
        ],
        out_specs=pl.BlockSpec((2, BE, H), lambda i: (0, i, 0)),
        out_shape=jax.ShapeDtypeStruct((2, E, H), jnp.float32),
    )(G, d2, vecs, W2, W2p)


def _tc_update(h, pe_h, Aa, Ab, V1, c1, V2, c2, P1, p1, P2, p2,
               nxt=None):
    N = h.shape[0]
    BN = 2000
    fused = nxt is not None

    def body(h_ref, pe_ref, aa_ref, ab_ref, v1, c1r, v2, c2r, q1, p1r, q2, p2r,
             *rest):
        if fused:
            wt, wu, b1r, bp1r, hn_ref, pen_ref, t_ref, u_ref = rest
        else:
            hn_ref, pen_ref = rest
        a = aa_ref[0] + ab_ref[0]
        ap = aa_ref[1] + ab_ref[1]
        cat = jnp.concatenate([h_ref[...], pe_ref[...], a], axis=1)
        z = _silu(jnp.dot(cat, v1[...], preferred_element_type=jnp.float32) + c1r[...])
        upd = jnp.dot(z, v2[...], preferred_element_type=jnp.float32) + c2r[...]
        hn = h_ref[...] + upd
        hn_ref[...] = hn
        catp = jnp.concatenate([pe_ref[...], ap], axis=1)
        zp = jnp.tanh(jnp.dot(catp, q1[...], preferred_element_type=jnp.float32) + p1r[...])
        updp = jnp.tanh(jnp.dot(zp, q2[...], preferred_element_type=jnp.float32) + p2r[...])
        pen = pe_ref[...] + updp
        pen_ref[...] = pen
        if fused:
            zz = jnp.concatenate([hn, pen], axis=1)
            tm = jnp.dot(zz, wt[:, :H], preferred_element_type=jnp.float32) + b1r[...]
            tpp = jnp.dot(zz, wt[:, H:], preferred_element_type=jnp.float32) + bp1r[...]
            um = jnp.dot(zz, wu[:, :H], preferred_element_type=jnp.float32)
            up = jnp.dot(zz, wu[:, H:], preferred_element_type=jnp.float32)
            t_ref[...] = _pack_bf16_pair(tm, tpp)
            u_ref[...] = _pack_bf16_pair(um, up)

    full = lambda s: pl.BlockSpec(s, lambda i: (0, 0))
    row = lambda: pl.BlockSpec((BN, H), lambda i: (i, 0))
    arow = lambda: pl.BlockSpec((2, BN, H), lambda i: (0, i, 0))
    in_specs = [
        row(), row(), arow(), arow(),
        full(V1.shape), full((1, H)), full(V2.shape), full((1, H)),
        full(P1.shape), full((1, H)), full(P2.shape), full((1, H)),
    ]
    args = [h, pe_h, Aa, Ab, V1, c1[None, :], V2, c2[None, :],
            P1, p1[None, :], P2, p2[None, :]]
    out_specs = [row(), row()]
    out_shape = [jax.ShapeDtypeStruct((N, H), jnp.float32),
                 jax.ShapeDtypeStruct((N, H), jnp.float32)]
    if fused:
        WT, WU, b1, bp1 = nxt
        in_specs += [full(WT.shape), full(WU.shape), full((1, H)), full((1, H))]
        args += [WT, WU, b1[None, :], bp1[None, :]]
        out_specs += [row(), row()]
        out_shape += [jax.ShapeDtypeStruct((N, H), jnp.uint32),
                      jax.ShapeDtypeStruct((N, H), jnp.uint32)]
    return pl.pallas_call(
        body,
        grid=(N // BN,),
        in_specs=in_specs,
        out_specs=out_specs,
        out_shape=out_shape,
    )(*args)


def _tc_final(h, batch2d, NB, Q1, q1, Q2, q2, R1, r1, R2p, r2p):
    N = h.shape[0]

    def body(h_ref, b_ref, w1, b1r, w2, b2r, w3, b3r, w4, b4r, out_ref):
        t = _silu(jnp.dot(h_ref[...], w1[...], preferred_element_type=jnp.float32)
                  + b1r[...])
        hpre = jnp.dot(t, w2[...], preferred_element_type=jnp.float32) + b2r[...]
        seg = lax.broadcasted_iota(jnp.int32, (NB, N), 0)
        oh = (b_ref[...] == seg).astype(jnp.float32)
        pooled = jnp.dot(oh, hpre, preferred_element_type=jnp.float32)
        tr = _silu(jnp.dot(pooled, w3[...], preferred_element_type=jnp.float32)
                   + b3r[...])
        out_ref[...] = jnp.dot(tr, w4[...], preferred_element_type=jnp.float32) + b4r[...]

    return pl.pallas_call(
        body,
        out_shape=jax.ShapeDtypeStruct((NB, H), jnp.float32),
    )(h, batch2d, Q1, q1[None, :], Q2, q2[None, :],
      R1, r1[None, :], R2p, r2p[None, :])


def kernel(x, pos, pe, params, edge_index, batch):
    N = x.shape[0]
    E = edge_index.shape[1]
    NB = 64

    send = edge_index[0].astype(jnp.int32)
    rec = edge_index[1].astype(jnp.int32)

    d2 = _sc_d2(pos[:, 0], pos[:, 1], pos[:, 2], send, rec)
    dist = _tc_dist(d2.reshape(E // H, H)).reshape(E, 1)

    layers = params['layers']
    L = len(layers)
    WTs, WUs, vecss, b1s, bp1s = [], [], [], [], []
    zblk = jnp.zeros((H, H), jnp.float32)
    for lp in layers:
        W1 = lp['message_mlp'][0]['w']
        b1 = lp['message_mlp'][0]['b']
        b2 = lp['message_mlp'][1]['b']
        Wp1 = lp['message_pos_mlp'][0]['w']
        bp1 = lp['message_pos_mlp'][0]['b']
        bp2 = lp['message_pos_mlp'][1]['b']
        WTs.append(jnp.concatenate([
            jnp.concatenate([W1[:H], zblk], axis=1),
            jnp.concatenate([W1[H:2 * H], Wp1[:H]], axis=1)], axis=0))
        WUs.append(jnp.concatenate([
            jnp.concatenate([W1[2 * H:3 * H], zblk], axis=1),
            jnp.concatenate([W1[3 * H:4 * H], Wp1[H:2 * H]], axis=1)], axis=0))
        zrow = jnp.zeros((H,), jnp.float32)
        vecss.append(jnp.stack([W1[4 * H], Wp1[2 * H], b2, bp2,
                                zrow, zrow, zrow, zrow], axis=0))
        b1s.append(b1)
        bp1s.append(bp1)

    emb = params['embed']
    embp = params['embed_pe']
    h, pe_h, T2, U2 = _tc_embed(
        x, pe,
        emb[0]['w'], emb[0]['b'], emb[1]['w'], emb[1]['b'],
        embp[0]['w'], embp[0]['b'], embp[1]['w'], embp[1]['b'],
        WTs[0], WUs[0], b1s[0], bp1s[0])

    zeros_nh = jnp.zeros((10240, H), jnp.float32)

    E2 = E // 2
    eidxA = jnp.concatenate([send[:E2], rec[:E2]])
    eidxB = jnp.concatenate([send[E2:], rec[E2:]])
    recA, recB = rec[:E2], rec[E2:]
    distA, distB = dist[:E2], dist[E2:]
    NPAD = 10240

    def _tables(t2, u2):
        tu = jnp.stack([t2, u2], axis=0)
        return jnp.pad(tu, ((0, 0), (0, NPAD - N), (0, 0)))

    for li, lp in enumerate(layers):
        W2 = lp['message_mlp'][1]['w']
        Wp2 = lp['message_pos_mlp'][1]['w']
        TU = _tables(T2, U2)
        GA = _sc_gather(TU, eidxA)
        MA = _tc_edge(GA, distA, vecss[li], W2, Wp2)
        GB = _sc_gather(TU, eidxB)
        Aa = _sc_scatter(MA, recA, zeros_nh)
        MB = _tc_edge(GB, distB, vecss[li], W2, Wp2)
        Ab = _sc_scatter(MB, recB, zeros_nh)

        nxt = None
        if li + 1 < L:
            nxt = (WTs[li + 1], WUs[li + 1], b1s[li + 1], bp1s[li + 1])
        res = _tc_update(h, pe_h, Aa[:, :N], Ab[:, :N],
                         lp['update_mlp'][0]['w'], lp['update_mlp'][0]['b'],
                         lp['update_mlp'][1]['w'], lp['update_mlp'][1]['b'],
                         lp['update_pos_mlp'][0]['w'], lp['update_pos_mlp'][0]['b'],
                         lp['update_pos_mlp'][1]['w'], lp['update_pos_mlp'][1]['b'],
                         nxt=nxt)
        if nxt is None:
            h, pe_h = res
        else:
            h, pe_h, T2, U2 = res

    pr = params['pre_readout']
    ro = params['readout']
    R2 = ro[1]['w']
    R2p = jnp.concatenate([R2, jnp.zeros((H, H - 1), jnp.float32)], axis=1)
    r2p = jnp.concatenate([ro[1]['b'], jnp.zeros((H - 1,), jnp.float32)], axis=0)
    batch2d = batch.astype(jnp.int32)[None, :]
    out = _tc_final(h, batch2d, NB,
                    pr[0]['w'], pr[0]['b'], pr[1]['w'], pr[1]['b'],
                    ro[0]['w'], ro[0]['b'], R2p, r2p)
    return out[:, 0]
